# Initial kernel scaffold; baseline (speedup 1.0000x reference)
import jax
import jax.numpy as jnp
from jax.experimental import pallas as pl


def kernel(feat_stage1, feat_stage2, feat_stage3, feat_stage5, intrinsics, extrinsics, params):
    raise NotImplementedError("write your pallas kernel here")


# speedup vs baseline: 6.9663x; 6.9663x over previous
import functools

import jax
import jax.numpy as jnp
from jax import lax
from jax.experimental import pallas as pl
from jax.experimental.pallas import tpu as pltpu
from jax.experimental.pallas import tpu_sc as plsc

_SCALES = ["stage1", "stage2", "stage3", "stage5"]
_HWS = {"stage1": (128, 128), "stage2": (64, 64), "stage3": (32, 32), "stage5": (16, 16)}
_SKIP = ["stage1", "stage2", "stage3"]
_DEPTH_CH = 64
_BEV_H = 128
_BEV_W = 128
_NCELL = _BEV_H * _BEV_W
_VOX = (0.4, 0.4)

_CH = 128
_HALF = _NCELL // 2
_DUMP = _HALF
_ACC_ROWS = _HALF + 128
_ZROWS = 130


@functools.cache
def _make_sc_scatter(N, B):
    mesh = plsc.VectorSubcoreMesh(core_axis_name="c", subcore_axis_name="s")
    chunk = N // 16
    K = min(128, chunk)
    nk = chunk // K

    @functools.partial(
        pl.kernel,
        mesh=mesh,
        out_type=jax.ShapeDtypeStruct((B, _NCELL, _CH), jnp.float32),
        scratch_types=[
            pltpu.VMEM_SHARED((_ACC_ROWS, _CH), jnp.float32),
            pltpu.VMEM((K,), jnp.int32),
            pltpu.VMEM((K, _CH), jnp.float32),
            pltpu.VMEM((_ZROWS, _CH), jnp.float32),
            pltpu.VMEM((128, _CH), jnp.float32),
        ],
    )
    def kfn(idx_hbm, rows_hbm, acc_hbm, acc_sh, idx_v, rows_v, zbuf, wbuf):
        c = lax.axis_index("c")
        s = lax.axis_index("s")
        half_base = c * _HALF

        def zrow(i, _):
            def zcol(j, __):
                zbuf[i, pl.ds(j * 16, 16)] = jnp.zeros((16,), jnp.float32)
                return 0
            return lax.fori_loop(0, _CH // 16, zcol, 0)
        lax.fori_loop(0, _ZROWS, zrow, 0)

        for b in range(B):
            for j in range(4):
                pltpu.sync_copy(zbuf, acc_sh.at[pl.ds(s * 4 * _ZROWS + j * _ZROWS, _ZROWS)])
            plsc.subcore_barrier()

            def body(t, _):
                off = s * chunk + t * K
                pltpu.sync_copy(idx_hbm.at[b, pl.ds(off, K)], idx_v)
                pltpu.sync_copy(rows_hbm.at[b, pl.ds(off, K)], rows_v)
                for v in range(K // 16):
                    t16 = idx_v[pl.ds(v * 16, 16)]
                    loc = t16 - half_base
                    ok = (loc >= 0) & (loc < _HALF)
                    idx_v[pl.ds(v * 16, 16)] = jnp.where(ok, loc, _DUMP)
                pltpu.sync_copy(rows_v, acc_sh.at[idx_v], add=True)
                return 0
            lax.fori_loop(0, nk, body, 0)
            plsc.subcore_barrier()

            for j in range(4):
                r0 = s * 512 + j * 128
                pltpu.sync_copy(acc_sh.at[pl.ds(r0, 128)], wbuf)
                pltpu.sync_copy(wbuf, acc_hbm.at[b, pl.ds(half_base + r0, 128)])
            plsc.subcore_barrier()

    return kfn


@functools.cache
def _make_tc_transpose(N, B):
    NB = min(512, N)

    def body(x_ref, o_ref):
        o_ref[...] = jnp.transpose(x_ref[...], (0, 2, 1))

    return pl.pallas_call(
        body,
        grid=(B, N // NB),
        in_specs=[pl.BlockSpec((1, _CH, NB), lambda b, n: (b, 0, n))],
        out_specs=pl.BlockSpec((1, NB, _CH), lambda b, n: (b, n, 0)),
        out_shape=jax.ShapeDtypeStruct((B, N, _CH), jnp.float32),
    )


def _conv2d(x, w, bias=None, padding=0, groups=1):
    out = lax.conv_general_dilated(
        x, w, window_strides=(1, 1),
        padding=[(padding, padding), (padding, padding)],
        feature_group_count=groups,
        dimension_numbers=("NCHW", "OIHW", "NCHW"))
    if bias is not None:
        out = out + bias[None, :, None, None]
    return out


def _bnorm(x, g, b, eps=1e-5):
    m = jnp.mean(x, axis=(0, 2, 3), keepdims=True)
    v = jnp.var(x, axis=(0, 2, 3), keepdims=True)
    return (x - m) / jnp.sqrt(v + eps) * g[None, :, None, None] + b[None, :, None, None]


def _pixel_grid(H, W):
    x = jnp.linspace(0.0, W - 1.0, W)
    y = jnp.linspace(0.0, H - 1.0, H)
    yy, xx = jnp.meshgrid(y, x, indexing="ij")
    return jnp.stack([xx, yy, jnp.ones_like(xx)], axis=-1).reshape(-1, 3).T


def kernel(feat_stage1, feat_stage2, feat_stage3, feat_stage5, intrinsics, extrinsics, params):
    feats = {"stage1": feat_stage1, "stage2": feat_stage2, "stage3": feat_stage3, "stage5": feat_stage5}
    Bsz = feat_stage1.shape[0]
    depth_bins = jnp.exp(jnp.linspace(jnp.log(1.0), jnp.log(60.0), _DEPTH_CH))
    K_inv = jnp.linalg.inv(intrinsics)

    accs = {}
    for s in _SCALES:
        p = params[s]
        f = feats[s]
        H, W = f.shape[2], f.shape[3]
        reduced = jax.nn.relu(_bnorm(_conv2d(f, p["fr_w1"]), p["fr_g1"], p["fr_b1"]))
        reduced = jax.nn.relu(_bnorm(_conv2d(reduced, p["fr_w2"], padding=1, groups=8), p["fr_g2"], p["fr_b2"]))
        h = jax.nn.relu(_bnorm(_conv2d(f, p["dn_w1"]), p["dn_g1"], p["dn_b1"]))
        depth_logits = _conv2d(h, p["dn_w2"], bias=p["dn_bias2"])
        depth_probs = jax.nn.softmax(depth_logits * 10.0, axis=1)
        depth_map = jnp.sum(depth_probs * depth_bins[None, :, None, None], axis=1)
        c = jax.nn.relu(_bnorm(_conv2d(jnp.concatenate([depth_logits, reduced], axis=1), p["cn_w1"], padding=1), p["cn_g1"], p["cn_b1"]))
        confidence = jax.nn.sigmoid(_conv2d(c, p["cn_w2"], bias=p["cn_bias2"]))
        pixels = jnp.broadcast_to(_pixel_grid(H, W)[None], (Bsz, 3, H * W))
        dm = depth_map.reshape(Bsz, 1, -1)
        cam_pts = dm * jnp.einsum("bij,bjn->bin", K_inv, pixels)
        cam_pts_h = jnp.concatenate([cam_pts, jnp.ones_like(cam_pts[:, :1])], axis=1)
        ego = jnp.einsum("bij,bjn->bin", extrinsics, cam_pts_h)[:, :3]
        bev_x = (ego[:, 0] / _VOX[0] + _BEV_W // 2).astype(jnp.int32)
        bev_y = (ego[:, 1] / _VOX[1] + _BEV_H // 2).astype(jnp.int32)
        valid = (bev_x >= 0) & (bev_x < _BEV_W) & (bev_y >= 0) & (bev_y < _BEV_H)
        weighted = jnp.where(valid[:, None, :], reduced.reshape(Bsz, _CH, -1) * confidence.reshape(Bsz, 1, -1), 0.0)
        idx = jnp.where(valid, bev_y * _BEV_W + bev_x, 0)

        rows = _make_tc_transpose(H * W, Bsz)(weighted)
        accs[s] = _make_sc_scatter(H * W, Bsz)(idx.astype(jnp.int32), rows)

    outs = []
    for s in _SKIP:
        sp = params["sp_" + s]
        acc = jnp.transpose(accs[s], (0, 2, 1)).reshape(Bsz, _CH, _BEV_H, _BEV_W)
        outs.append(jax.nn.relu(_bnorm(_conv2d(acc, sp["w"]), sp["g"], sp["b"])))
    mp = params["mp"]
    main_acc = jnp.transpose(accs["stage5"], (0, 2, 1)).reshape(Bsz, _CH, _BEV_H, _BEV_W)
    main = jax.nn.relu(_bnorm(_conv2d(main_acc, mp["w"], padding=1), mp["g"], mp["b"]))
    return (main, outs[0], outs[1], outs[2])


# speedup vs baseline: 7.4640x; 1.0714x over previous
import functools

import jax
import jax.numpy as jnp
from jax import lax
from jax.experimental import pallas as pl
from jax.experimental.pallas import tpu as pltpu
from jax.experimental.pallas import tpu_sc as plsc

_SCALES = ["stage1", "stage2", "stage3", "stage5"]
_HWS = {"stage1": (128, 128), "stage2": (64, 64), "stage3": (32, 32), "stage5": (16, 16)}
_SKIP = ["stage1", "stage2", "stage3"]
_DEPTH_CH = 64
_BEV_H = 128
_BEV_W = 128
_NCELL = _BEV_H * _BEV_W
_VOX = (0.4, 0.4)

_CH = 128
_HALF = _NCELL // 2
_DUMP = _HALF
_ACC_ROWS = _HALF + 128
_ZROWS = 130


@functools.cache
def _make_sc_scatter(N, B):
    mesh = plsc.VectorSubcoreMesh(core_axis_name="c", subcore_axis_name="s")
    chunk = N // 16
    K = min(128, chunk)
    nk = chunk // K

    @functools.partial(
        pl.kernel,
        mesh=mesh,
        out_type=jax.ShapeDtypeStruct((B, _NCELL, _CH), jnp.float32),
        scratch_types=[
            pltpu.VMEM_SHARED((_ACC_ROWS, _CH), jnp.float32),
            pltpu.VMEM((K,), jnp.int32),
            pltpu.VMEM((K, _CH), jnp.float32),
            pltpu.VMEM((_ZROWS, _CH), jnp.float32),
            pltpu.VMEM((128, _CH), jnp.float32),
        ],
    )
    def kfn(idx_hbm, rows_hbm, acc_hbm, acc_sh, idx_v, rows_v, zbuf, wbuf):
        c = lax.axis_index("c")
        s = lax.axis_index("s")
        half_base = c * _HALF

        def zrow(i, _):
            def zcol(j, __):
                zbuf[i, pl.ds(j * 16, 16)] = jnp.zeros((16,), jnp.float32)
                return 0
            return lax.fori_loop(0, _CH // 16, zcol, 0)
        lax.fori_loop(0, _ZROWS, zrow, 0)

        for b in range(B):
            for j in range(4):
                pltpu.sync_copy(zbuf, acc_sh.at[pl.ds(s * 4 * _ZROWS + j * _ZROWS, _ZROWS)])
            plsc.subcore_barrier()

            def body(t, _):
                off = s * chunk + t * K
                pltpu.sync_copy(idx_hbm.at[b, pl.ds(off, K)], idx_v)
                pltpu.sync_copy(rows_hbm.at[b, pl.ds(off, K)], rows_v)
                for v in range(K // 16):
                    t16 = idx_v[pl.ds(v * 16, 16)]
                    loc = t16 - half_base
                    ok = (loc >= 0) & (loc < _HALF)
                    idx_v[pl.ds(v * 16, 16)] = jnp.where(ok, loc, _DUMP)
                pltpu.sync_copy(rows_v, acc_sh.at[idx_v], add=True)
                return 0
            lax.fori_loop(0, nk, body, 0)
            plsc.subcore_barrier()

            for j in range(4):
                r0 = s * 512 + j * 128
                pltpu.sync_copy(acc_sh.at[pl.ds(r0, 128)], wbuf)
                pltpu.sync_copy(wbuf, acc_hbm.at[b, pl.ds(half_base + r0, 128)])
            plsc.subcore_barrier()

    return kfn


@functools.cache
def _make_tc_transpose(N, B):
    NB = min(512, N)

    def body(x_ref, o_ref):
        o_ref[...] = jnp.transpose(x_ref[...], (0, 2, 1))

    return pl.pallas_call(
        body,
        grid=(B, N // NB),
        in_specs=[pl.BlockSpec((1, _CH, NB), lambda b, n: (b, 0, n))],
        out_specs=pl.BlockSpec((1, NB, _CH), lambda b, n: (b, n, 0)),
        out_shape=jax.ShapeDtypeStruct((B, N, _CH), jnp.float32),
    )


_BN_EPS = 1e-5


_NSTRIP = 8
_STRIP = _NCELL // _NSTRIP
_SROWS = _STRIP // _BEV_W


@functools.cache
def _make_skip_head(co, B):
    inv_n = 1.0 / (B * _NCELL)

    def body(x_ref, w_ref, g_ref, b_ref, o_ref, m_sc, s_sc):
        p = pl.program_id(0)
        b = pl.program_id(1)
        i = pl.program_id(2)

        @pl.when(jnp.logical_and(p == 0, jnp.logical_and(b == 0, i == 0)))
        def _():
            m_sc[...] = jnp.zeros_like(m_sc)
            s_sc[...] = jnp.zeros_like(s_sc)

        xb = x_ref[0].astype(jnp.bfloat16)
        wb = w_ref[...].astype(jnp.bfloat16)
        o = jax.lax.dot_general(xb, wb, (((1,), (1,)), ((), ())),
                                preferred_element_type=jnp.float32)

        @pl.when(p == 0)
        def _():
            m_sc[...] += jnp.sum(o, axis=0, keepdims=True)
            s_sc[...] += jnp.sum(o * o, axis=0, keepdims=True)

        @pl.when(p == 1)
        def _():
            mean = m_sc[...] * inv_n
            var = s_sc[...] * inv_n - mean * mean
            scale = g_ref[...] / jnp.sqrt(var + _BN_EPS)
            shift = b_ref[...] - mean * scale
            y = jax.nn.relu(o * scale + shift)
            o_ref[0] = jnp.transpose(y, (1, 0)).reshape(co, _SROWS, _BEV_W)

    return pl.pallas_call(
        body,
        grid=(2, B, _NSTRIP),
        in_specs=[
            pl.BlockSpec((1, _STRIP, _CH), lambda p, b, i: (b, i, 0)),
            pl.BlockSpec((co, _CH), lambda p, b, i: (0, 0)),
            pl.BlockSpec((1, co), lambda p, b, i: (0, 0)),
            pl.BlockSpec((1, co), lambda p, b, i: (0, 0)),
        ],
        out_specs=pl.BlockSpec((1, co, _SROWS, _BEV_W), lambda p, b, i: (b, 0, i, 0)),
        out_shape=jax.ShapeDtypeStruct((B, co, _BEV_H, _BEV_W), jnp.float32),
        scratch_shapes=[
            pltpu.VMEM((1, co), jnp.float32),
            pltpu.VMEM((1, co), jnp.float32),
        ],
    )


@functools.cache
def _make_main_head(B):
    inv_n = 1.0 / (B * _NCELL)

    def body(x_ref, up_ref, dn_ref, w_ref, g_ref, b_ref, o_ref, m_sc, s_sc):
        p = pl.program_id(0)
        b = pl.program_id(1)
        i = pl.program_id(2)

        @pl.when(jnp.logical_and(p == 0, jnp.logical_and(b == 0, i == 0)))
        def _():
            m_sc[...] = jnp.zeros_like(m_sc)
            s_sc[...] = jnp.zeros_like(s_sc)

        above = up_ref[0].astype(jnp.bfloat16) * (i > 0).astype(jnp.bfloat16)
        below = dn_ref[0].astype(jnp.bfloat16) * (i < _NSTRIP - 1).astype(jnp.bfloat16)
        xb = jnp.concatenate([above, x_ref[0].astype(jnp.bfloat16), below], axis=0)

        wv = w_ref[...].astype(jnp.bfloat16)
        wq = jax.lax.broadcasted_iota(jnp.int32, (_STRIP, 1), 0) % _BEV_W
        o = jnp.zeros((_STRIP, _CH), jnp.float32)
        for kx in range(3):
            a = jnp.zeros((_STRIP, _CH), jnp.float32)
            for ky in range(3):
                src = jax.lax.slice(xb, (ky * _BEV_W, 0), (ky * _BEV_W + _STRIP, _CH))
                a = a + jax.lax.dot_general(src, wv[ky, kx], (((1,), (0,)), ((), ())),
                                            preferred_element_type=jnp.float32)
            if kx == 0:
                sh = jnp.concatenate([jnp.zeros((1, _CH), jnp.float32), a[:-1]], axis=0)
                o = o + sh * (wq >= 1).astype(jnp.float32)
            elif kx == 1:
                o = o + a
            else:
                sh = jnp.concatenate([a[1:], jnp.zeros((1, _CH), jnp.float32)], axis=0)
                o = o + sh * (wq <= _BEV_W - 2).astype(jnp.float32)

        @pl.when(p == 0)
        def _():
            m_sc[...] += jnp.sum(o, axis=0, keepdims=True)
            s_sc[...] += jnp.sum(o * o, axis=0, keepdims=True)

        @pl.when(p == 1)
        def _():
            mean = m_sc[...] * inv_n
            var = s_sc[...] * inv_n - mean * mean
            scale = g_ref[...] / jnp.sqrt(var + _BN_EPS)
            shift = b_ref[...] - mean * scale
            y = jax.nn.relu(o * scale + shift)
            o_ref[0] = jnp.transpose(y, (1, 0)).reshape(_CH, _SROWS, _BEV_W)

    nrow = _NCELL // _BEV_W
    return pl.pallas_call(
        body,
        grid=(2, B, _NSTRIP),
        in_specs=[
            pl.BlockSpec((1, _STRIP, _CH), lambda p, b, i: (b, i, 0)),
            pl.BlockSpec((1, _BEV_W, _CH),
                         lambda p, b, i: (b, jnp.maximum(i * _SROWS - 1, 0), 0)),
            pl.BlockSpec((1, _BEV_W, _CH),
                         lambda p, b, i: (b, jnp.minimum((i + 1) * _SROWS, nrow - 1), 0)),
            pl.BlockSpec((3, 3, _CH, _CH), lambda p, b, i: (0, 0, 0, 0)),
            pl.BlockSpec((1, _CH), lambda p, b, i: (0, 0)),
            pl.BlockSpec((1, _CH), lambda p, b, i: (0, 0)),
        ],
        out_specs=pl.BlockSpec((1, _CH, _SROWS, _BEV_W), lambda p, b, i: (b, 0, i, 0)),
        out_shape=jax.ShapeDtypeStruct((B, _CH, _BEV_H, _BEV_W), jnp.float32),
        scratch_shapes=[
            pltpu.VMEM((1, _CH), jnp.float32),
            pltpu.VMEM((1, _CH), jnp.float32),
        ],
    )


def _conv2d(x, w, bias=None, padding=0, groups=1):
    out = lax.conv_general_dilated(
        x, w, window_strides=(1, 1),
        padding=[(padding, padding), (padding, padding)],
        feature_group_count=groups,
        dimension_numbers=("NCHW", "OIHW", "NCHW"))
    if bias is not None:
        out = out + bias[None, :, None, None]
    return out


def _bnorm(x, g, b, eps=1e-5):
    m = jnp.mean(x, axis=(0, 2, 3), keepdims=True)
    v = jnp.var(x, axis=(0, 2, 3), keepdims=True)
    return (x - m) / jnp.sqrt(v + eps) * g[None, :, None, None] + b[None, :, None, None]


def _pixel_grid(H, W):
    x = jnp.linspace(0.0, W - 1.0, W)
    y = jnp.linspace(0.0, H - 1.0, H)
    yy, xx = jnp.meshgrid(y, x, indexing="ij")
    return jnp.stack([xx, yy, jnp.ones_like(xx)], axis=-1).reshape(-1, 3).T


def kernel(feat_stage1, feat_stage2, feat_stage3, feat_stage5, intrinsics, extrinsics, params):
    feats = {"stage1": feat_stage1, "stage2": feat_stage2, "stage3": feat_stage3, "stage5": feat_stage5}
    Bsz = feat_stage1.shape[0]
    depth_bins = jnp.exp(jnp.linspace(jnp.log(1.0), jnp.log(60.0), _DEPTH_CH))
    K_inv = jnp.linalg.inv(intrinsics)

    accs = {}
    for s in _SCALES:
        p = params[s]
        f = feats[s]
        H, W = f.shape[2], f.shape[3]
        reduced = jax.nn.relu(_bnorm(_conv2d(f, p["fr_w1"]), p["fr_g1"], p["fr_b1"]))
        reduced = jax.nn.relu(_bnorm(_conv2d(reduced, p["fr_w2"], padding=1, groups=8), p["fr_g2"], p["fr_b2"]))
        h = jax.nn.relu(_bnorm(_conv2d(f, p["dn_w1"]), p["dn_g1"], p["dn_b1"]))
        depth_logits = _conv2d(h, p["dn_w2"], bias=p["dn_bias2"])
        depth_probs = jax.nn.softmax(depth_logits * 10.0, axis=1)
        depth_map = jnp.sum(depth_probs * depth_bins[None, :, None, None], axis=1)
        c = jax.nn.relu(_bnorm(_conv2d(jnp.concatenate([depth_logits, reduced], axis=1), p["cn_w1"], padding=1), p["cn_g1"], p["cn_b1"]))
        confidence = jax.nn.sigmoid(_conv2d(c, p["cn_w2"], bias=p["cn_bias2"]))
        pixels = jnp.broadcast_to(_pixel_grid(H, W)[None], (Bsz, 3, H * W))
        dm = depth_map.reshape(Bsz, 1, -1)
        cam_pts = dm * jnp.einsum("bij,bjn->bin", K_inv, pixels)
        cam_pts_h = jnp.concatenate([cam_pts, jnp.ones_like(cam_pts[:, :1])], axis=1)
        ego = jnp.einsum("bij,bjn->bin", extrinsics, cam_pts_h)[:, :3]
        bev_x = (ego[:, 0] / _VOX[0] + _BEV_W // 2).astype(jnp.int32)
        bev_y = (ego[:, 1] / _VOX[1] + _BEV_H // 2).astype(jnp.int32)
        valid = (bev_x >= 0) & (bev_x < _BEV_W) & (bev_y >= 0) & (bev_y < _BEV_H)
        weighted = jnp.where(valid[:, None, :], reduced.reshape(Bsz, _CH, -1) * confidence.reshape(Bsz, 1, -1), 0.0)
        idx = jnp.where(valid, bev_y * _BEV_W + bev_x, 0)

        rows = _make_tc_transpose(H * W, Bsz)(weighted)
        accs[s] = _make_sc_scatter(H * W, Bsz)(idx.astype(jnp.int32), rows)

    outs = []
    for s in _SKIP:
        sp = params["sp_" + s]
        co = sp["w"].shape[0]
        outs.append(_make_skip_head(co, Bsz)(
            accs[s], sp["w"].reshape(co, _CH),
            sp["g"].reshape(1, co), sp["b"].reshape(1, co)))
    mp = params["mp"]
    main = _make_main_head(Bsz)(
        accs["stage5"], accs["stage5"], accs["stage5"],
        jnp.transpose(mp["w"], (2, 3, 1, 0)),
        mp["g"].reshape(1, _CH), mp["b"].reshape(1, _CH))
    return (main, outs[0], outs[1], outs[2])


# speedup vs baseline: 13.2493x; 1.7751x over previous
import functools

import jax
import jax.numpy as jnp
from jax import lax
from jax.experimental import pallas as pl
from jax.experimental.pallas import tpu as pltpu
from jax.experimental.pallas import tpu_sc as plsc

_SCALES = ["stage1", "stage2", "stage3", "stage5"]
_HWS = {"stage1": (128, 128), "stage2": (64, 64), "stage3": (32, 32), "stage5": (16, 16)}
_SKIP = ["stage1", "stage2", "stage3"]
_DEPTH_CH = 64
_BEV_H = 128
_BEV_W = 128
_NCELL = _BEV_H * _BEV_W
_VOX = (0.4, 0.4)

_CH = 128
_HALF = _NCELL // 2
_DUMP = _HALF
_ACC_ROWS = _HALF + 128
_ZROWS = 130


@functools.cache
def _make_sc_scatter(N, B):
    mesh = plsc.VectorSubcoreMesh(core_axis_name="c", subcore_axis_name="s")
    chunk = N // 16
    K = min(128, chunk)
    nk = chunk // K

    @functools.partial(
        pl.kernel,
        mesh=mesh,
        out_type=jax.ShapeDtypeStruct((B, _NCELL, _CH), jnp.float32),
        scratch_types=[
            pltpu.VMEM_SHARED((_ACC_ROWS, _CH), jnp.float32),
            pltpu.VMEM((K,), jnp.int32),
            pltpu.VMEM((K, _CH), jnp.float32),
            pltpu.VMEM((_ZROWS, _CH), jnp.float32),
            pltpu.VMEM((128, _CH), jnp.float32),
        ],
    )
    def kfn(idx_hbm, rows_hbm, acc_hbm, acc_sh, idx_v, rows_v, zbuf, wbuf):
        c = lax.axis_index("c")
        s = lax.axis_index("s")
        half_base = c * _HALF

        def zrow(i, _):
            def zcol(j, __):
                zbuf[i, pl.ds(j * 16, 16)] = jnp.zeros((16,), jnp.float32)
                return 0
            return lax.fori_loop(0, _CH // 16, zcol, 0)
        lax.fori_loop(0, _ZROWS, zrow, 0)

        for b in range(B):
            for j in range(4):
                pltpu.sync_copy(zbuf, acc_sh.at[pl.ds(s * 4 * _ZROWS + j * _ZROWS, _ZROWS)])
            plsc.subcore_barrier()

            def body(t, _):
                off = s * chunk + t * K
                pltpu.sync_copy(idx_hbm.at[b, pl.ds(off, K)], idx_v)
                pltpu.sync_copy(rows_hbm.at[b, pl.ds(off, K)], rows_v)
                for v in range(K // 16):
                    t16 = idx_v[pl.ds(v * 16, 16)]
                    loc = t16 - half_base
                    ok = (loc >= 0) & (loc < _HALF)
                    idx_v[pl.ds(v * 16, 16)] = jnp.where(ok, loc, _DUMP)
                pltpu.sync_copy(rows_v, acc_sh.at[idx_v], add=True)
                return 0
            lax.fori_loop(0, nk, body, 0)
            plsc.subcore_barrier()

            for j in range(4):
                r0 = s * 512 + j * 128
                pltpu.sync_copy(acc_sh.at[pl.ds(r0, 128)], wbuf)
                pltpu.sync_copy(wbuf, acc_hbm.at[b, pl.ds(half_base + r0, 128)])
            plsc.subcore_barrier()

    return kfn


def _strip_cfg(H, W):
    srows = min(16, H)
    return srows, srows * W, H // srows


def _conv3x3_taps(xbs, wv, strip, W, CO):
    wq = jax.lax.broadcasted_iota(jnp.int32, (strip, 1), 0) % W
    o = jnp.zeros((strip, CO), jnp.float32)
    for kx in range(3):
        a = jnp.zeros((strip, CO), jnp.float32)
        for ky in range(3):
            for xb, w in zip(xbs, wv):
                src = jax.lax.slice(xb, (ky * W, 0), (ky * W + strip, xb.shape[1]))
                a = a + jax.lax.dot_general(src, w[ky, kx], (((1,), (0,)), ((), ())),
                                            preferred_element_type=jnp.float32)
        if kx == 0:
            sh = jnp.concatenate([jnp.zeros((1, CO), jnp.float32), a[:-1]], axis=0)
            o = o + sh * (wq >= 1).astype(jnp.float32)
        elif kx == 1:
            o = o + a
        else:
            sh = jnp.concatenate([a[1:], jnp.zeros((1, CO), jnp.float32)], axis=0)
            o = o + sh * (wq <= W - 2).astype(jnp.float32)
    return o


def _halo_specs(H, W, CI, srows, strip):
    return [
        pl.BlockSpec((1, strip, CI), lambda b, i: (b, i, 0)),
        pl.BlockSpec((1, W, CI), lambda b, i: (b, jnp.maximum(i * srows - 1, 0), 0)),
        pl.BlockSpec((1, W, CI), lambda b, i: (b, jnp.minimum((i + 1) * srows, H - 1), 0)),
    ]


def _edge_halo(up, dn, i, ns):
    above = up * (i > 0).astype(up.dtype)
    below = dn * (i < ns - 1).astype(dn.dtype)
    return above, below


@functools.cache
def _make_fr2(H, W, B):
    srows, strip, ns = _strip_cfg(H, W)

    def body(x_ref, up_ref, dn_ref, w_ref, v_ref, st_ref):
        b = pl.program_id(0)
        i = pl.program_id(1)

        @pl.when(jnp.logical_and(b == 0, i == 0))
        def _():
            st_ref[...] = jnp.zeros_like(st_ref)

        above, below = _edge_halo(up_ref[0].astype(jnp.bfloat16),
                                  dn_ref[0].astype(jnp.bfloat16), i, ns)
        xb = jnp.concatenate([above, x_ref[0].astype(jnp.bfloat16), below], axis=0)
        o = _conv3x3_taps([xb], [w_ref[...].astype(jnp.bfloat16)], strip, W, _CH)
        v_ref[0] = o
        st_ref[0, :] += jnp.sum(o, axis=0)
        st_ref[1, :] += jnp.sum(o * o, axis=0)

    return pl.pallas_call(
        body,
        grid=(B, ns),
        in_specs=_halo_specs(H, W, 64, srows, strip) + [
            pl.BlockSpec((3, 3, 64, _CH), lambda b, i: (0, 0, 0, 0)),
        ],
        out_specs=[
            pl.BlockSpec((1, strip, _CH), lambda b, i: (b, i, 0)),
            pl.BlockSpec((2, _CH), lambda b, i: (0, 0)),
        ],
        out_shape=[
            jax.ShapeDtypeStruct((B, H * W, _CH), jnp.float32),
            jax.ShapeDtypeStruct((2, _CH), jnp.float32),
        ],
    )


@functools.cache
def _make_cn1(H, W, B):
    srows, strip, ns = _strip_cfg(H, W)
    inv_n = 1.0 / (B * H * W)

    def body(dl_ref, dlu_ref, dld_ref, v_ref, vu_ref, vd_ref,
             st_v_ref, g_ref, bb_ref, wdl_ref, wrd_ref, o_ref, st_ref):
        b = pl.program_id(0)
        i = pl.program_id(1)

        @pl.when(jnp.logical_and(b == 0, i == 0))
        def _():
            st_ref[...] = jnp.zeros_like(st_ref)

        mean = st_v_ref[0:1, :] * inv_n
        var = st_v_ref[1:2, :] * inv_n - mean * mean
        scale = g_ref[...] / jnp.sqrt(var + _BN_EPS)
        shift = bb_ref[...] - mean * scale

        def red(x):
            return jax.nn.relu(x * scale + shift).astype(jnp.bfloat16)

        vau, vbd = _edge_halo(red(vu_ref[0]), red(vd_ref[0]), i, ns)
        vb = jnp.concatenate([vau, red(v_ref[0]), vbd], axis=0)
        dau, dbd = _edge_halo(dlu_ref[0].astype(jnp.bfloat16),
                              dld_ref[0].astype(jnp.bfloat16), i, ns)
        db = jnp.concatenate([dau, dl_ref[0].astype(jnp.bfloat16), dbd], axis=0)

        o = _conv3x3_taps([db, vb],
                          [wdl_ref[...].astype(jnp.bfloat16),
                           wrd_ref[...].astype(jnp.bfloat16)], strip, W, 64)
        o_ref[0] = o
        st_ref[0, :] += jnp.sum(o, axis=0)
        st_ref[1, :] += jnp.sum(o * o, axis=0)

    return pl.pallas_call(
        body,
        grid=(B, ns),
        in_specs=(_halo_specs(H, W, 64, srows, strip)
                  + _halo_specs(H, W, _CH, srows, strip)
                  + [
                      pl.BlockSpec((2, _CH), lambda b, i: (0, 0)),
                      pl.BlockSpec((1, _CH), lambda b, i: (0, 0)),
                      pl.BlockSpec((1, _CH), lambda b, i: (0, 0)),
                      pl.BlockSpec((3, 3, 64, 64), lambda b, i: (0, 0, 0, 0)),
                      pl.BlockSpec((3, 3, _CH, 64), lambda b, i: (0, 0, 0, 0)),
                  ]),
        out_specs=[
            pl.BlockSpec((1, strip, 64), lambda b, i: (b, i, 0)),
            pl.BlockSpec((2, 64), lambda b, i: (0, 0)),
        ],
        out_shape=[
            jax.ShapeDtypeStruct((B, H * W, 64), jnp.float32),
            jax.ShapeDtypeStruct((2, 64), jnp.float32),
        ],
    )


@functools.cache
def _make_rows(H, W, B):
    srows, strip, ns = _strip_cfg(H, W)
    inv_n = 1.0 / (B * H * W)

    def body(wp_ref, st_w_ref, cg_ref, cb_ref, w2_ref, b2_ref,
             v_ref, st_v_ref, g_ref, bb_ref, valid_ref, o_ref):
        meanw = st_w_ref[0:1, :] * inv_n
        varw = st_w_ref[1:2, :] * inv_n - meanw * meanw
        scw = cg_ref[...] / jnp.sqrt(varw + _BN_EPS)
        shw = cb_ref[...] - meanw * scw
        c = jax.nn.relu(wp_ref[0] * scw + shw).astype(jnp.bfloat16)
        w2 = w2_ref[...].astype(jnp.bfloat16).astype(jnp.float32)
        logit = jnp.sum(c.astype(jnp.float32) * w2, axis=1, keepdims=True)
        conf = jax.nn.sigmoid(logit + b2_ref[...])

        meanv = st_v_ref[0:1, :] * inv_n
        varv = st_v_ref[1:2, :] * inv_n - meanv * meanv
        scv = g_ref[...] / jnp.sqrt(varv + _BN_EPS)
        shv = bb_ref[...] - meanv * scv
        reduced = jax.nn.relu(v_ref[0] * scv + shv)
        o_ref[0] = reduced * conf * valid_ref[0, :, 0:1]

    return pl.pallas_call(
        body,
        grid=(B, ns),
        in_specs=[
            pl.BlockSpec((1, strip, 64), lambda b, i: (b, i, 0)),
            pl.BlockSpec((2, 64), lambda b, i: (0, 0)),
            pl.BlockSpec((1, 64), lambda b, i: (0, 0)),
            pl.BlockSpec((1, 64), lambda b, i: (0, 0)),
            pl.BlockSpec((1, 64), lambda b, i: (0, 0)),
            pl.BlockSpec((1, 1), lambda b, i: (0, 0)),
            pl.BlockSpec((1, strip, _CH), lambda b, i: (b, i, 0)),
            pl.BlockSpec((2, _CH), lambda b, i: (0, 0)),
            pl.BlockSpec((1, _CH), lambda b, i: (0, 0)),
            pl.BlockSpec((1, _CH), lambda b, i: (0, 0)),
            pl.BlockSpec((1, strip, 1), lambda b, i: (b, i, 0)),
        ],
        out_specs=pl.BlockSpec((1, strip, _CH), lambda b, i: (b, i, 0)),
        out_shape=jax.ShapeDtypeStruct((B, H * W, _CH), jnp.float32),
    )


_BN_EPS = 1e-5


_NSTRIP = 8
_STRIP = _NCELL // _NSTRIP
_SROWS = _STRIP // _BEV_W


@functools.cache
def _make_skip_head(co, B):
    inv_n = 1.0 / (B * _NCELL)

    def body(x_ref, w_ref, g_ref, b_ref, o_ref, m_sc, s_sc):
        p = pl.program_id(0)
        b = pl.program_id(1)
        i = pl.program_id(2)

        @pl.when(jnp.logical_and(p == 0, jnp.logical_and(b == 0, i == 0)))
        def _():
            m_sc[...] = jnp.zeros_like(m_sc)
            s_sc[...] = jnp.zeros_like(s_sc)

        xb = x_ref[0].astype(jnp.bfloat16)
        wb = w_ref[...].astype(jnp.bfloat16)
        o = jax.lax.dot_general(xb, wb, (((1,), (1,)), ((), ())),
                                preferred_element_type=jnp.float32)

        @pl.when(p == 0)
        def _():
            m_sc[...] += jnp.sum(o, axis=0, keepdims=True)
            s_sc[...] += jnp.sum(o * o, axis=0, keepdims=True)

        @pl.when(p == 1)
        def _():
            mean = m_sc[...] * inv_n
            var = s_sc[...] * inv_n - mean * mean
            scale = g_ref[...] / jnp.sqrt(var + _BN_EPS)
            shift = b_ref[...] - mean * scale
            y = jax.nn.relu(o * scale + shift)
            o_ref[0] = jnp.transpose(y, (1, 0)).reshape(co, _SROWS, _BEV_W)

    return pl.pallas_call(
        body,
        grid=(2, B, _NSTRIP),
        in_specs=[
            pl.BlockSpec((1, _STRIP, _CH), lambda p, b, i: (b, i, 0)),
            pl.BlockSpec((co, _CH), lambda p, b, i: (0, 0)),
            pl.BlockSpec((1, co), lambda p, b, i: (0, 0)),
            pl.BlockSpec((1, co), lambda p, b, i: (0, 0)),
        ],
        out_specs=pl.BlockSpec((1, co, _SROWS, _BEV_W), lambda p, b, i: (b, 0, i, 0)),
        out_shape=jax.ShapeDtypeStruct((B, co, _BEV_H, _BEV_W), jnp.float32),
        scratch_shapes=[
            pltpu.VMEM((1, co), jnp.float32),
            pltpu.VMEM((1, co), jnp.float32),
        ],
    )


@functools.cache
def _make_main_head(B):
    inv_n = 1.0 / (B * _NCELL)

    def body(x_ref, up_ref, dn_ref, w_ref, g_ref, b_ref, o_ref, m_sc, s_sc):
        p = pl.program_id(0)
        b = pl.program_id(1)
        i = pl.program_id(2)

        @pl.when(jnp.logical_and(p == 0, jnp.logical_and(b == 0, i == 0)))
        def _():
            m_sc[...] = jnp.zeros_like(m_sc)
            s_sc[...] = jnp.zeros_like(s_sc)

        above = up_ref[0].astype(jnp.bfloat16) * (i > 0).astype(jnp.bfloat16)
        below = dn_ref[0].astype(jnp.bfloat16) * (i < _NSTRIP - 1).astype(jnp.bfloat16)
        xb = jnp.concatenate([above, x_ref[0].astype(jnp.bfloat16), below], axis=0)

        wv = w_ref[...].astype(jnp.bfloat16)
        wq = jax.lax.broadcasted_iota(jnp.int32, (_STRIP, 1), 0) % _BEV_W
        o = jnp.zeros((_STRIP, _CH), jnp.float32)
        for kx in range(3):
            a = jnp.zeros((_STRIP, _CH), jnp.float32)
            for ky in range(3):
                src = jax.lax.slice(xb, (ky * _BEV_W, 0), (ky * _BEV_W + _STRIP, _CH))
                a = a + jax.lax.dot_general(src, wv[ky, kx], (((1,), (0,)), ((), ())),
                                            preferred_element_type=jnp.float32)
            if kx == 0:
                sh = jnp.concatenate([jnp.zeros((1, _CH), jnp.float32), a[:-1]], axis=0)
                o = o + sh * (wq >= 1).astype(jnp.float32)
            elif kx == 1:
                o = o + a
            else:
                sh = jnp.concatenate([a[1:], jnp.zeros((1, _CH), jnp.float32)], axis=0)
                o = o + sh * (wq <= _BEV_W - 2).astype(jnp.float32)

        @pl.when(p == 0)
        def _():
            m_sc[...] += jnp.sum(o, axis=0, keepdims=True)
            s_sc[...] += jnp.sum(o * o, axis=0, keepdims=True)

        @pl.when(p == 1)
        def _():
            mean = m_sc[...] * inv_n
            var = s_sc[...] * inv_n - mean * mean
            scale = g_ref[...] / jnp.sqrt(var + _BN_EPS)
            shift = b_ref[...] - mean * scale
            y = jax.nn.relu(o * scale + shift)
            o_ref[0] = jnp.transpose(y, (1, 0)).reshape(_CH, _SROWS, _BEV_W)

    nrow = _NCELL // _BEV_W
    return pl.pallas_call(
        body,
        grid=(2, B, _NSTRIP),
        in_specs=[
            pl.BlockSpec((1, _STRIP, _CH), lambda p, b, i: (b, i, 0)),
            pl.BlockSpec((1, _BEV_W, _CH),
                         lambda p, b, i: (b, jnp.maximum(i * _SROWS - 1, 0), 0)),
            pl.BlockSpec((1, _BEV_W, _CH),
                         lambda p, b, i: (b, jnp.minimum((i + 1) * _SROWS, nrow - 1), 0)),
            pl.BlockSpec((3, 3, _CH, _CH), lambda p, b, i: (0, 0, 0, 0)),
            pl.BlockSpec((1, _CH), lambda p, b, i: (0, 0)),
            pl.BlockSpec((1, _CH), lambda p, b, i: (0, 0)),
        ],
        out_specs=pl.BlockSpec((1, _CH, _SROWS, _BEV_W), lambda p, b, i: (b, 0, i, 0)),
        out_shape=jax.ShapeDtypeStruct((B, _CH, _BEV_H, _BEV_W), jnp.float32),
        scratch_shapes=[
            pltpu.VMEM((1, _CH), jnp.float32),
            pltpu.VMEM((1, _CH), jnp.float32),
        ],
    )


def _conv2d(x, w, bias=None, padding=0, groups=1):
    out = lax.conv_general_dilated(
        x, w, window_strides=(1, 1),
        padding=[(padding, padding), (padding, padding)],
        feature_group_count=groups,
        dimension_numbers=("NCHW", "OIHW", "NCHW"))
    if bias is not None:
        out = out + bias[None, :, None, None]
    return out


def _bnorm(x, g, b, eps=1e-5):
    m = jnp.mean(x, axis=(0, 2, 3), keepdims=True)
    v = jnp.var(x, axis=(0, 2, 3), keepdims=True)
    return (x - m) / jnp.sqrt(v + eps) * g[None, :, None, None] + b[None, :, None, None]


def _pixel_grid(H, W):
    x = jnp.linspace(0.0, W - 1.0, W)
    y = jnp.linspace(0.0, H - 1.0, H)
    yy, xx = jnp.meshgrid(y, x, indexing="ij")
    return jnp.stack([xx, yy, jnp.ones_like(xx)], axis=-1).reshape(-1, 3).T


def kernel(feat_stage1, feat_stage2, feat_stage3, feat_stage5, intrinsics, extrinsics, params):
    feats = {"stage1": feat_stage1, "stage2": feat_stage2, "stage3": feat_stage3, "stage5": feat_stage5}
    Bsz = feat_stage1.shape[0]
    depth_bins = jnp.exp(jnp.linspace(jnp.log(1.0), jnp.log(60.0), _DEPTH_CH))
    K_inv = jnp.linalg.inv(intrinsics)

    accs = {}
    for s in _SCALES:
        p = params[s]
        f = feats[s]
        H, W = f.shape[2], f.shape[3]
        N = H * W

        h = jax.nn.relu(_bnorm(_conv2d(f, p["dn_w1"]), p["dn_g1"], p["dn_b1"]))
        depth_logits = _conv2d(h, p["dn_w2"], bias=p["dn_bias2"])
        depth_probs = jax.nn.softmax(depth_logits * 10.0, axis=1)
        depth_map = jnp.sum(depth_probs * depth_bins[None, :, None, None], axis=1)
        pixels = jnp.broadcast_to(_pixel_grid(H, W)[None], (Bsz, 3, H * W))
        dm = depth_map.reshape(Bsz, 1, -1)
        cam_pts = dm * jnp.einsum("bij,bjn->bin", K_inv, pixels)
        cam_pts_h = jnp.concatenate([cam_pts, jnp.ones_like(cam_pts[:, :1])], axis=1)
        ego = jnp.einsum("bij,bjn->bin", extrinsics, cam_pts_h)[:, :3]
        bev_x = (ego[:, 0] / _VOX[0] + _BEV_W // 2).astype(jnp.int32)
        bev_y = (ego[:, 1] / _VOX[1] + _BEV_H // 2).astype(jnp.int32)
        valid = (bev_x >= 0) & (bev_x < _BEV_W) & (bev_y >= 0) & (bev_y < _BEV_H)
        idx = jnp.where(valid, bev_y * _BEV_W + bev_x, 0)

        r1 = jax.nn.relu(_bnorm(_conv2d(f, p["fr_w1"]), p["fr_g1"], p["fr_b1"]))
        r1p = jnp.transpose(r1.reshape(Bsz, 64, N), (0, 2, 1))
        w2d = jnp.zeros((_CH, 64, 3, 3), jnp.float32)
        for g in range(8):
            w2d = w2d.at[g * 16:(g + 1) * 16, g * 8:(g + 1) * 8].set(
                p["fr_w2"][g * 16:(g + 1) * 16])
        w2d = jnp.transpose(w2d, (2, 3, 1, 0))
        v, st_v = _make_fr2(H, W, Bsz)(r1p, r1p, r1p, w2d)

        dlp = jnp.transpose(depth_logits.reshape(Bsz, 64, N), (0, 2, 1))
        wdl = jnp.transpose(p["cn_w1"][:, :64], (2, 3, 1, 0))
        wrd = jnp.transpose(p["cn_w1"][:, 64:], (2, 3, 1, 0))
        g2 = p["fr_g2"].reshape(1, _CH)
        b2 = p["fr_b2"].reshape(1, _CH)
        wp, st_w = _make_cn1(H, W, Bsz)(
            dlp, dlp, dlp, v, v, v, st_v, g2, b2, wdl, wrd)

        rows = _make_rows(H, W, Bsz)(
            wp, st_w, p["cn_g1"].reshape(1, 64), p["cn_b1"].reshape(1, 64),
            p["cn_w2"].reshape(1, 64), p["cn_bias2"].reshape(1, 1),
            v, st_v, g2, b2,
            valid.astype(jnp.float32).reshape(Bsz, N, 1))
        accs[s] = _make_sc_scatter(H * W, Bsz)(idx.astype(jnp.int32), rows)

    outs = []
    for s in _SKIP:
        sp = params["sp_" + s]
        co = sp["w"].shape[0]
        outs.append(_make_skip_head(co, Bsz)(
            accs[s], sp["w"].reshape(co, _CH),
            sp["g"].reshape(1, co), sp["b"].reshape(1, co)))
    mp = params["mp"]
    main = _make_main_head(Bsz)(
        accs["stage5"], accs["stage5"], accs["stage5"],
        jnp.transpose(mp["w"], (2, 3, 1, 0)),
        mp["g"].reshape(1, _CH), mp["b"].reshape(1, _CH))
    return (main, outs[0], outs[1], outs[2])


# speedup vs baseline: 13.2886x; 1.0030x over previous
import functools

import jax
import jax.numpy as jnp
from jax import lax
from jax.experimental import pallas as pl
from jax.experimental.pallas import tpu as pltpu
from jax.experimental.pallas import tpu_sc as plsc

_SCALES = ["stage1", "stage2", "stage3", "stage5"]
_HWS = {"stage1": (128, 128), "stage2": (64, 64), "stage3": (32, 32), "stage5": (16, 16)}
_SKIP = ["stage1", "stage2", "stage3"]
_DEPTH_CH = 64
_BEV_H = 128
_BEV_W = 128
_NCELL = _BEV_H * _BEV_W
_VOX = (0.4, 0.4)

_CH = 128
_HALF = _NCELL // 2
_DUMP = _HALF
_ACC_ROWS = _HALF + 128
_ZROWS = 130


@functools.cache
def _make_sc_scatter(N, B):
    mesh = plsc.VectorSubcoreMesh(core_axis_name="c", subcore_axis_name="s")
    chunk = N // 16
    K = min(128, chunk)
    nk = chunk // K

    @functools.partial(
        pl.kernel,
        mesh=mesh,
        out_type=jax.ShapeDtypeStruct((B, _NCELL, _CH), jnp.float32),
        scratch_types=[
            pltpu.VMEM_SHARED((_ACC_ROWS, _CH), jnp.float32),
            pltpu.VMEM((K,), jnp.int32),
            pltpu.VMEM((K,), jnp.int32),
            pltpu.VMEM((K, _CH), jnp.float32),
            pltpu.VMEM((K, _CH), jnp.float32),
            pltpu.VMEM((_ZROWS, _CH), jnp.float32),
            pltpu.SemaphoreType.DMA,
            pltpu.SemaphoreType.DMA,
            pltpu.SemaphoreType.DMA,
            pltpu.SemaphoreType.DMA,
            pltpu.SemaphoreType.DMA,
        ],
    )
    def kfn(idx_hbm, rows_hbm, acc_hbm, acc_sh, idx0, idx1, rows0, rows1,
            zbuf, sl0, sl1, sc0, sc1, sw):
        c = lax.axis_index("c")
        s = lax.axis_index("s")
        half_base = c * _HALF
        idx_v = [idx0, idx1]
        rows_v = [rows0, rows1]
        sem_ld = [sl0, sl1]
        sem_sc = [sc0, sc1]

        def zrow(i, _):
            def zcol(j, __):
                zbuf[i, pl.ds(j * 16, 16)] = jnp.zeros((16,), jnp.float32)
                return 0
            return lax.fori_loop(0, _CH // 16, zcol, 0)
        lax.fori_loop(0, _ZROWS, zrow, 0)

        def load(b, t, p):
            off = s * chunk + t * K
            ha = pltpu.async_copy(idx_hbm.at[b, pl.ds(off, K)], idx_v[p], sem_ld[p])
            hb = pltpu.async_copy(rows_hbm.at[b, pl.ds(off, K)], rows_v[p], sem_ld[p])
            return ha, hb

        for b in range(B):
            hz = [pltpu.async_copy(
                      zbuf, acc_sh.at[pl.ds(s * 4 * _ZROWS + j * _ZROWS, _ZROWS)], sw)
                  for j in range(4)]
            for h in hz:
                h.wait()
            plsc.subcore_barrier()

            pend = load(b, 0, 0)
            hsc = [None, None]
            for t in range(nk):
                p = t % 2
                for h in pend:
                    h.wait()
                for v in range(K // 16):
                    t16 = idx_v[p][pl.ds(v * 16, 16)]
                    loc = t16 - half_base
                    ok = (loc >= 0) & (loc < _HALF)
                    idx_v[p][pl.ds(v * 16, 16)] = jnp.where(ok, loc, _DUMP)
                hsc[p] = pltpu.async_copy(rows_v[p], acc_sh.at[idx_v[p]],
                                          sem_sc[p], add=True)
                if t + 1 < nk:
                    if hsc[1 - p] is not None:
                        hsc[1 - p].wait()
                        hsc[1 - p] = None
                    pend = load(b, t + 1, 1 - p)
            for p in range(2):
                if hsc[p] is not None:
                    hsc[p].wait()
            plsc.subcore_barrier()

            hw = [pltpu.async_copy(
                      acc_sh.at[pl.ds(s * 512 + j * 128, 128)],
                      acc_hbm.at[b, pl.ds(half_base + s * 512 + j * 128, 128)], sw)
                  for j in range(4)]
            for h in hw:
                h.wait()
            plsc.subcore_barrier()

    return kfn


def _strip_cfg(H, W):
    srows = min(16, H)
    return srows, srows * W, H // srows


def _conv3x3_taps(xbs, wv, strip, W, CO):
    wq = jax.lax.broadcasted_iota(jnp.int32, (strip, 1), 0) % W
    o = jnp.zeros((strip, CO), jnp.float32)
    for kx in range(3):
        a = jnp.zeros((strip, CO), jnp.float32)
        for ky in range(3):
            for xb, w in zip(xbs, wv):
                src = jax.lax.slice(xb, (ky * W, 0), (ky * W + strip, xb.shape[1]))
                a = a + jax.lax.dot_general(src, w[ky, kx], (((1,), (0,)), ((), ())),
                                            preferred_element_type=jnp.float32)
        if kx == 0:
            sh = jnp.concatenate([jnp.zeros((1, CO), jnp.float32), a[:-1]], axis=0)
            o = o + sh * (wq >= 1).astype(jnp.float32)
        elif kx == 1:
            o = o + a
        else:
            sh = jnp.concatenate([a[1:], jnp.zeros((1, CO), jnp.float32)], axis=0)
            o = o + sh * (wq <= W - 2).astype(jnp.float32)
    return o


def _halo_specs(H, W, CI, srows, strip):
    return [
        pl.BlockSpec((1, strip, CI), lambda b, i: (b, i, 0)),
        pl.BlockSpec((1, W, CI), lambda b, i: (b, jnp.maximum(i * srows - 1, 0), 0)),
        pl.BlockSpec((1, W, CI), lambda b, i: (b, jnp.minimum((i + 1) * srows, H - 1), 0)),
    ]


def _edge_halo(up, dn, i, ns):
    above = up * (i > 0).astype(up.dtype)
    below = dn * (i < ns - 1).astype(dn.dtype)
    return above, below


@functools.cache
def _make_fr2(H, W, B):
    srows, strip, ns = _strip_cfg(H, W)

    def body(x_ref, up_ref, dn_ref, w_ref, v_ref, st_ref):
        b = pl.program_id(0)
        i = pl.program_id(1)

        @pl.when(jnp.logical_and(b == 0, i == 0))
        def _():
            st_ref[...] = jnp.zeros_like(st_ref)

        above, below = _edge_halo(up_ref[0].astype(jnp.bfloat16),
                                  dn_ref[0].astype(jnp.bfloat16), i, ns)
        xb = jnp.concatenate([above, x_ref[0].astype(jnp.bfloat16), below], axis=0)
        o = _conv3x3_taps([xb], [w_ref[...].astype(jnp.bfloat16)], strip, W, _CH)
        v_ref[0] = o
        st_ref[0, :] += jnp.sum(o, axis=0)
        st_ref[1, :] += jnp.sum(o * o, axis=0)

    return pl.pallas_call(
        body,
        grid=(B, ns),
        in_specs=_halo_specs(H, W, 64, srows, strip) + [
            pl.BlockSpec((3, 3, 64, _CH), lambda b, i: (0, 0, 0, 0)),
        ],
        out_specs=[
            pl.BlockSpec((1, strip, _CH), lambda b, i: (b, i, 0)),
            pl.BlockSpec((2, _CH), lambda b, i: (0, 0)),
        ],
        out_shape=[
            jax.ShapeDtypeStruct((B, H * W, _CH), jnp.float32),
            jax.ShapeDtypeStruct((2, _CH), jnp.float32),
        ],
    )


@functools.cache
def _make_cn1(H, W, B):
    srows, strip, ns = _strip_cfg(H, W)
    inv_n = 1.0 / (B * H * W)

    def body(dl_ref, dlu_ref, dld_ref, v_ref, vu_ref, vd_ref,
             st_v_ref, g_ref, bb_ref, wdl_ref, wrd_ref, o_ref, st_ref):
        b = pl.program_id(0)
        i = pl.program_id(1)

        @pl.when(jnp.logical_and(b == 0, i == 0))
        def _():
            st_ref[...] = jnp.zeros_like(st_ref)

        mean = st_v_ref[0:1, :] * inv_n
        var = st_v_ref[1:2, :] * inv_n - mean * mean
        scale = g_ref[...] / jnp.sqrt(var + _BN_EPS)
        shift = bb_ref[...] - mean * scale

        def red(x):
            return jax.nn.relu(x * scale + shift).astype(jnp.bfloat16)

        vau, vbd = _edge_halo(red(vu_ref[0]), red(vd_ref[0]), i, ns)
        vb = jnp.concatenate([vau, red(v_ref[0]), vbd], axis=0)
        dau, dbd = _edge_halo(dlu_ref[0].astype(jnp.bfloat16),
                              dld_ref[0].astype(jnp.bfloat16), i, ns)
        db = jnp.concatenate([dau, dl_ref[0].astype(jnp.bfloat16), dbd], axis=0)

        o = _conv3x3_taps([db, vb],
                          [wdl_ref[...].astype(jnp.bfloat16),
                           wrd_ref[...].astype(jnp.bfloat16)], strip, W, 64)
        o_ref[0] = o
        st_ref[0, :] += jnp.sum(o, axis=0)
        st_ref[1, :] += jnp.sum(o * o, axis=0)

    return pl.pallas_call(
        body,
        grid=(B, ns),
        in_specs=(_halo_specs(H, W, 64, srows, strip)
                  + _halo_specs(H, W, _CH, srows, strip)
                  + [
                      pl.BlockSpec((2, _CH), lambda b, i: (0, 0)),
                      pl.BlockSpec((1, _CH), lambda b, i: (0, 0)),
                      pl.BlockSpec((1, _CH), lambda b, i: (0, 0)),
                      pl.BlockSpec((3, 3, 64, 64), lambda b, i: (0, 0, 0, 0)),
                      pl.BlockSpec((3, 3, _CH, 64), lambda b, i: (0, 0, 0, 0)),
                  ]),
        out_specs=[
            pl.BlockSpec((1, strip, 64), lambda b, i: (b, i, 0)),
            pl.BlockSpec((2, 64), lambda b, i: (0, 0)),
        ],
        out_shape=[
            jax.ShapeDtypeStruct((B, H * W, 64), jnp.float32),
            jax.ShapeDtypeStruct((2, 64), jnp.float32),
        ],
    )


@functools.cache
def _make_rows(H, W, B):
    srows, strip, ns = _strip_cfg(H, W)
    inv_n = 1.0 / (B * H * W)

    def body(wp_ref, st_w_ref, cg_ref, cb_ref, w2_ref, b2_ref,
             v_ref, st_v_ref, g_ref, bb_ref, valid_ref, o_ref):
        meanw = st_w_ref[0:1, :] * inv_n
        varw = st_w_ref[1:2, :] * inv_n - meanw * meanw
        scw = cg_ref[...] / jnp.sqrt(varw + _BN_EPS)
        shw = cb_ref[...] - meanw * scw
        c = jax.nn.relu(wp_ref[0] * scw + shw).astype(jnp.bfloat16)
        w2 = w2_ref[...].astype(jnp.bfloat16).astype(jnp.float32)
        logit = jnp.sum(c.astype(jnp.float32) * w2, axis=1, keepdims=True)
        conf = jax.nn.sigmoid(logit + b2_ref[...])

        meanv = st_v_ref[0:1, :] * inv_n
        varv = st_v_ref[1:2, :] * inv_n - meanv * meanv
        scv = g_ref[...] / jnp.sqrt(varv + _BN_EPS)
        shv = bb_ref[...] - meanv * scv
        reduced = jax.nn.relu(v_ref[0] * scv + shv)
        o_ref[0] = reduced * conf * valid_ref[0, :, 0:1]

    return pl.pallas_call(
        body,
        grid=(B, ns),
        in_specs=[
            pl.BlockSpec((1, strip, 64), lambda b, i: (b, i, 0)),
            pl.BlockSpec((2, 64), lambda b, i: (0, 0)),
            pl.BlockSpec((1, 64), lambda b, i: (0, 0)),
            pl.BlockSpec((1, 64), lambda b, i: (0, 0)),
            pl.BlockSpec((1, 64), lambda b, i: (0, 0)),
            pl.BlockSpec((1, 1), lambda b, i: (0, 0)),
            pl.BlockSpec((1, strip, _CH), lambda b, i: (b, i, 0)),
            pl.BlockSpec((2, _CH), lambda b, i: (0, 0)),
            pl.BlockSpec((1, _CH), lambda b, i: (0, 0)),
            pl.BlockSpec((1, _CH), lambda b, i: (0, 0)),
            pl.BlockSpec((1, strip, 1), lambda b, i: (b, i, 0)),
        ],
        out_specs=pl.BlockSpec((1, strip, _CH), lambda b, i: (b, i, 0)),
        out_shape=jax.ShapeDtypeStruct((B, H * W, _CH), jnp.float32),
    )


_BN_EPS = 1e-5


_NSTRIP = 8
_STRIP = _NCELL // _NSTRIP
_SROWS = _STRIP // _BEV_W


@functools.cache
def _make_skip_head(co, B):
    inv_n = 1.0 / (B * _NCELL)

    def body(x_ref, w_ref, g_ref, b_ref, o_ref, m_sc, s_sc):
        p = pl.program_id(0)
        b = pl.program_id(1)
        i = pl.program_id(2)

        @pl.when(jnp.logical_and(p == 0, jnp.logical_and(b == 0, i == 0)))
        def _():
            m_sc[...] = jnp.zeros_like(m_sc)
            s_sc[...] = jnp.zeros_like(s_sc)

        xb = x_ref[0].astype(jnp.bfloat16)
        wb = w_ref[...].astype(jnp.bfloat16)
        o = jax.lax.dot_general(xb, wb, (((1,), (1,)), ((), ())),
                                preferred_element_type=jnp.float32)

        @pl.when(p == 0)
        def _():
            m_sc[...] += jnp.sum(o, axis=0, keepdims=True)
            s_sc[...] += jnp.sum(o * o, axis=0, keepdims=True)

        @pl.when(p == 1)
        def _():
            mean = m_sc[...] * inv_n
            var = s_sc[...] * inv_n - mean * mean
            scale = g_ref[...] / jnp.sqrt(var + _BN_EPS)
            shift = b_ref[...] - mean * scale
            y = jax.nn.relu(o * scale + shift)
            o_ref[0] = jnp.transpose(y, (1, 0)).reshape(co, _SROWS, _BEV_W)

    return pl.pallas_call(
        body,
        grid=(2, B, _NSTRIP),
        in_specs=[
            pl.BlockSpec((1, _STRIP, _CH), lambda p, b, i: (b, i, 0)),
            pl.BlockSpec((co, _CH), lambda p, b, i: (0, 0)),
            pl.BlockSpec((1, co), lambda p, b, i: (0, 0)),
            pl.BlockSpec((1, co), lambda p, b, i: (0, 0)),
        ],
        out_specs=pl.BlockSpec((1, co, _SROWS, _BEV_W), lambda p, b, i: (b, 0, i, 0)),
        out_shape=jax.ShapeDtypeStruct((B, co, _BEV_H, _BEV_W), jnp.float32),
        scratch_shapes=[
            pltpu.VMEM((1, co), jnp.float32),
            pltpu.VMEM((1, co), jnp.float32),
        ],
    )


@functools.cache
def _make_main_head(B):
    inv_n = 1.0 / (B * _NCELL)

    def body(x_ref, up_ref, dn_ref, w_ref, g_ref, b_ref, o_ref, m_sc, s_sc):
        p = pl.program_id(0)
        b = pl.program_id(1)
        i = pl.program_id(2)

        @pl.when(jnp.logical_and(p == 0, jnp.logical_and(b == 0, i == 0)))
        def _():
            m_sc[...] = jnp.zeros_like(m_sc)
            s_sc[...] = jnp.zeros_like(s_sc)

        above = up_ref[0].astype(jnp.bfloat16) * (i > 0).astype(jnp.bfloat16)
        below = dn_ref[0].astype(jnp.bfloat16) * (i < _NSTRIP - 1).astype(jnp.bfloat16)
        xb = jnp.concatenate([above, x_ref[0].astype(jnp.bfloat16), below], axis=0)

        wv = w_ref[...].astype(jnp.bfloat16)
        wq = jax.lax.broadcasted_iota(jnp.int32, (_STRIP, 1), 0) % _BEV_W
        o = jnp.zeros((_STRIP, _CH), jnp.float32)
        for kx in range(3):
            a = jnp.zeros((_STRIP, _CH), jnp.float32)
            for ky in range(3):
                src = jax.lax.slice(xb, (ky * _BEV_W, 0), (ky * _BEV_W + _STRIP, _CH))
                a = a + jax.lax.dot_general(src, wv[ky, kx], (((1,), (0,)), ((), ())),
                                            preferred_element_type=jnp.float32)
            if kx == 0:
                sh = jnp.concatenate([jnp.zeros((1, _CH), jnp.float32), a[:-1]], axis=0)
                o = o + sh * (wq >= 1).astype(jnp.float32)
            elif kx == 1:
                o = o + a
            else:
                sh = jnp.concatenate([a[1:], jnp.zeros((1, _CH), jnp.float32)], axis=0)
                o = o + sh * (wq <= _BEV_W - 2).astype(jnp.float32)

        @pl.when(p == 0)
        def _():
            m_sc[...] += jnp.sum(o, axis=0, keepdims=True)
            s_sc[...] += jnp.sum(o * o, axis=0, keepdims=True)

        @pl.when(p == 1)
        def _():
            mean = m_sc[...] * inv_n
            var = s_sc[...] * inv_n - mean * mean
            scale = g_ref[...] / jnp.sqrt(var + _BN_EPS)
            shift = b_ref[...] - mean * scale
            y = jax.nn.relu(o * scale + shift)
            o_ref[0] = jnp.transpose(y, (1, 0)).reshape(_CH, _SROWS, _BEV_W)

    nrow = _NCELL // _BEV_W
    return pl.pallas_call(
        body,
        grid=(2, B, _NSTRIP),
        in_specs=[
            pl.BlockSpec((1, _STRIP, _CH), lambda p, b, i: (b, i, 0)),
            pl.BlockSpec((1, _BEV_W, _CH),
                         lambda p, b, i: (b, jnp.maximum(i * _SROWS - 1, 0), 0)),
            pl.BlockSpec((1, _BEV_W, _CH),
                         lambda p, b, i: (b, jnp.minimum((i + 1) * _SROWS, nrow - 1), 0)),
            pl.BlockSpec((3, 3, _CH, _CH), lambda p, b, i: (0, 0, 0, 0)),
            pl.BlockSpec((1, _CH), lambda p, b, i: (0, 0)),
            pl.BlockSpec((1, _CH), lambda p, b, i: (0, 0)),
        ],
        out_specs=pl.BlockSpec((1, _CH, _SROWS, _BEV_W), lambda p, b, i: (b, 0, i, 0)),
        out_shape=jax.ShapeDtypeStruct((B, _CH, _BEV_H, _BEV_W), jnp.float32),
        scratch_shapes=[
            pltpu.VMEM((1, _CH), jnp.float32),
            pltpu.VMEM((1, _CH), jnp.float32),
        ],
    )


def _conv2d(x, w, bias=None, padding=0, groups=1):
    out = lax.conv_general_dilated(
        x, w, window_strides=(1, 1),
        padding=[(padding, padding), (padding, padding)],
        feature_group_count=groups,
        dimension_numbers=("NCHW", "OIHW", "NCHW"))
    if bias is not None:
        out = out + bias[None, :, None, None]
    return out


def _bnorm(x, g, b, eps=1e-5):
    m = jnp.mean(x, axis=(0, 2, 3), keepdims=True)
    v = jnp.var(x, axis=(0, 2, 3), keepdims=True)
    return (x - m) / jnp.sqrt(v + eps) * g[None, :, None, None] + b[None, :, None, None]


def _pixel_grid(H, W):
    x = jnp.linspace(0.0, W - 1.0, W)
    y = jnp.linspace(0.0, H - 1.0, H)
    yy, xx = jnp.meshgrid(y, x, indexing="ij")
    return jnp.stack([xx, yy, jnp.ones_like(xx)], axis=-1).reshape(-1, 3).T


def kernel(feat_stage1, feat_stage2, feat_stage3, feat_stage5, intrinsics, extrinsics, params):
    feats = {"stage1": feat_stage1, "stage2": feat_stage2, "stage3": feat_stage3, "stage5": feat_stage5}
    Bsz = feat_stage1.shape[0]
    depth_bins = jnp.exp(jnp.linspace(jnp.log(1.0), jnp.log(60.0), _DEPTH_CH))
    K_inv = jnp.linalg.inv(intrinsics)

    accs = {}
    for s in _SCALES:
        p = params[s]
        f = feats[s]
        H, W = f.shape[2], f.shape[3]
        N = H * W

        h = jax.nn.relu(_bnorm(_conv2d(f, p["dn_w1"]), p["dn_g1"], p["dn_b1"]))
        depth_logits = _conv2d(h, p["dn_w2"], bias=p["dn_bias2"])
        depth_probs = jax.nn.softmax(depth_logits * 10.0, axis=1)
        depth_map = jnp.sum(depth_probs * depth_bins[None, :, None, None], axis=1)
        pixels = jnp.broadcast_to(_pixel_grid(H, W)[None], (Bsz, 3, H * W))
        dm = depth_map.reshape(Bsz, 1, -1)
        cam_pts = dm * jnp.einsum("bij,bjn->bin", K_inv, pixels)
        cam_pts_h = jnp.concatenate([cam_pts, jnp.ones_like(cam_pts[:, :1])], axis=1)
        ego = jnp.einsum("bij,bjn->bin", extrinsics, cam_pts_h)[:, :3]
        bev_x = (ego[:, 0] / _VOX[0] + _BEV_W // 2).astype(jnp.int32)
        bev_y = (ego[:, 1] / _VOX[1] + _BEV_H // 2).astype(jnp.int32)
        valid = (bev_x >= 0) & (bev_x < _BEV_W) & (bev_y >= 0) & (bev_y < _BEV_H)
        idx = jnp.where(valid, bev_y * _BEV_W + bev_x, 0)

        r1 = jax.nn.relu(_bnorm(_conv2d(f, p["fr_w1"]), p["fr_g1"], p["fr_b1"]))
        r1p = jnp.transpose(r1.reshape(Bsz, 64, N), (0, 2, 1))
        w2d = jnp.zeros((_CH, 64, 3, 3), jnp.float32)
        for g in range(8):
            w2d = w2d.at[g * 16:(g + 1) * 16, g * 8:(g + 1) * 8].set(
                p["fr_w2"][g * 16:(g + 1) * 16])
        w2d = jnp.transpose(w2d, (2, 3, 1, 0))
        v, st_v = _make_fr2(H, W, Bsz)(r1p, r1p, r1p, w2d)

        dlp = jnp.transpose(depth_logits.reshape(Bsz, 64, N), (0, 2, 1))
        wdl = jnp.transpose(p["cn_w1"][:, :64], (2, 3, 1, 0))
        wrd = jnp.transpose(p["cn_w1"][:, 64:], (2, 3, 1, 0))
        g2 = p["fr_g2"].reshape(1, _CH)
        b2 = p["fr_b2"].reshape(1, _CH)
        wp, st_w = _make_cn1(H, W, Bsz)(
            dlp, dlp, dlp, v, v, v, st_v, g2, b2, wdl, wrd)

        rows = _make_rows(H, W, Bsz)(
            wp, st_w, p["cn_g1"].reshape(1, 64), p["cn_b1"].reshape(1, 64),
            p["cn_w2"].reshape(1, 64), p["cn_bias2"].reshape(1, 1),
            v, st_v, g2, b2,
            valid.astype(jnp.float32).reshape(Bsz, N, 1))
        accs[s] = _make_sc_scatter(H * W, Bsz)(idx.astype(jnp.int32), rows)

    outs = []
    for s in _SKIP:
        sp = params["sp_" + s]
        co = sp["w"].shape[0]
        outs.append(_make_skip_head(co, Bsz)(
            accs[s], sp["w"].reshape(co, _CH),
            sp["g"].reshape(1, co), sp["b"].reshape(1, co)))
    mp = params["mp"]
    main = _make_main_head(Bsz)(
        accs["stage5"], accs["stage5"], accs["stage5"],
        jnp.transpose(mp["w"], (2, 3, 1, 0)),
        mp["g"].reshape(1, _CH), mp["b"].reshape(1, _CH))
    return (main, outs[0], outs[1], outs[2])


# speedup vs baseline: 14.1741x; 1.0666x over previous
import functools

import jax
import jax.numpy as jnp
from jax import lax
from jax.experimental import pallas as pl
from jax.experimental.pallas import tpu as pltpu
from jax.experimental.pallas import tpu_sc as plsc

_SCALES = ["stage1", "stage2", "stage3", "stage5"]
_HWS = {"stage1": (128, 128), "stage2": (64, 64), "stage3": (32, 32), "stage5": (16, 16)}
_SKIP = ["stage1", "stage2", "stage3"]
_DEPTH_CH = 64
_BEV_H = 128
_BEV_W = 128
_NCELL = _BEV_H * _BEV_W
_VOX = (0.4, 0.4)

_CH = 128
_HALF = _NCELL // 2
_DUMP = _HALF
_ACC_ROWS = _HALF + 128
_ZROWS = 130


@functools.cache
def _make_sc_scatter(N, B):
    mesh = plsc.VectorSubcoreMesh(core_axis_name="c", subcore_axis_name="s")
    chunk = N // 16
    K = min(128, chunk)
    nk = chunk // K

    @functools.partial(
        pl.kernel,
        mesh=mesh,
        out_type=jax.ShapeDtypeStruct((B, _NCELL, _CH), jnp.float32),
        scratch_types=[
            pltpu.VMEM_SHARED((_ACC_ROWS, _CH), jnp.float32),
            pltpu.VMEM((K,), jnp.int32),
            pltpu.VMEM((K,), jnp.int32),
            pltpu.VMEM((K, _CH), jnp.float32),
            pltpu.VMEM((K, _CH), jnp.float32),
            pltpu.VMEM((_ZROWS, _CH), jnp.float32),
            pltpu.SemaphoreType.DMA,
            pltpu.SemaphoreType.DMA,
            pltpu.SemaphoreType.DMA,
            pltpu.SemaphoreType.DMA,
            pltpu.SemaphoreType.DMA,
        ],
    )
    def kfn(idx_hbm, rows_hbm, acc_hbm, acc_sh, idx0, idx1, rows0, rows1,
            zbuf, sl0, sl1, sc0, sc1, sw):
        c = lax.axis_index("c")
        s = lax.axis_index("s")
        half_base = c * _HALF
        idx_v = [idx0, idx1]
        rows_v = [rows0, rows1]
        sem_ld = [sl0, sl1]
        sem_sc = [sc0, sc1]

        def zrow(i, _):
            def zcol(j, __):
                zbuf[i, pl.ds(j * 16, 16)] = jnp.zeros((16,), jnp.float32)
                return 0
            return lax.fori_loop(0, _CH // 16, zcol, 0)
        lax.fori_loop(0, _ZROWS, zrow, 0)

        def load(b, t, p):
            off = s * chunk + t * K
            ha = pltpu.async_copy(idx_hbm.at[b, pl.ds(off, K)], idx_v[p], sem_ld[p])
            hb = pltpu.async_copy(rows_hbm.at[b, pl.ds(off, K)], rows_v[p], sem_ld[p])
            return ha, hb

        for b in range(B):
            hz = [pltpu.async_copy(
                      zbuf, acc_sh.at[pl.ds(s * 4 * _ZROWS + j * _ZROWS, _ZROWS)], sw)
                  for j in range(4)]
            for h in hz:
                h.wait()
            plsc.subcore_barrier()

            pend = load(b, 0, 0)
            hsc = [None, None]
            for t in range(nk):
                p = t % 2
                for h in pend:
                    h.wait()
                for v in range(K // 16):
                    t16 = idx_v[p][pl.ds(v * 16, 16)]
                    loc = t16 - half_base
                    ok = (loc >= 0) & (loc < _HALF)
                    idx_v[p][pl.ds(v * 16, 16)] = jnp.where(ok, loc, _DUMP)
                hsc[p] = pltpu.async_copy(rows_v[p], acc_sh.at[idx_v[p]],
                                          sem_sc[p], add=True)
                if t + 1 < nk:
                    if hsc[1 - p] is not None:
                        hsc[1 - p].wait()
                        hsc[1 - p] = None
                    pend = load(b, t + 1, 1 - p)
            for p in range(2):
                if hsc[p] is not None:
                    hsc[p].wait()
            plsc.subcore_barrier()

            hw = [pltpu.async_copy(
                      acc_sh.at[pl.ds(s * 512 + j * 128, 128)],
                      acc_hbm.at[b, pl.ds(half_base + s * 512 + j * 128, 128)], sw)
                  for j in range(4)]
            for h in hw:
                h.wait()
            plsc.subcore_barrier()

    return kfn


def _strip_cfg(H, W):
    srows = min(16, H)
    return srows, srows * W, H // srows


def _conv3x3_taps(xbs, wv, strip, W, CO):
    wq = jax.lax.broadcasted_iota(jnp.int32, (strip, 1), 0) % W
    o = jnp.zeros((strip, CO), jnp.float32)
    for kx in range(3):
        a = jnp.zeros((strip, CO), jnp.float32)
        for ky in range(3):
            for xb, w in zip(xbs, wv):
                src = jax.lax.slice(xb, (ky * W, 0), (ky * W + strip, xb.shape[1]))
                a = a + jax.lax.dot_general(src, w[ky, kx], (((1,), (0,)), ((), ())),
                                            preferred_element_type=jnp.float32)
        if kx == 0:
            sh = jnp.concatenate([jnp.zeros((1, CO), jnp.float32), a[:-1]], axis=0)
            o = o + sh * (wq >= 1).astype(jnp.float32)
        elif kx == 1:
            o = o + a
        else:
            sh = jnp.concatenate([a[1:], jnp.zeros((1, CO), jnp.float32)], axis=0)
            o = o + sh * (wq <= W - 2).astype(jnp.float32)
    return o


def _halo_specs(H, W, CI, srows, strip):
    return [
        pl.BlockSpec((1, strip, CI), lambda b, i: (b, i, 0)),
        pl.BlockSpec((1, W, CI), lambda b, i: (b, jnp.maximum(i * srows - 1, 0), 0)),
        pl.BlockSpec((1, W, CI), lambda b, i: (b, jnp.minimum((i + 1) * srows, H - 1), 0)),
    ]


def _edge_halo(up, dn, i, ns):
    above = up * (i > 0).astype(up.dtype)
    below = dn * (i < ns - 1).astype(dn.dtype)
    return above, below


@functools.cache
def _make_fr1(C, H, W, B):
    srows, strip, ns = _strip_cfg(H, W)

    def body(f_ref, w_ref, u_ref, st_ref):
        b = pl.program_id(0)
        i = pl.program_id(1)

        @pl.when(jnp.logical_and(b == 0, i == 0))
        def _():
            st_ref[...] = jnp.zeros_like(st_ref)

        fb = f_ref[0].astype(jnp.bfloat16)
        wb = w_ref[...].astype(jnp.bfloat16)
        o = jax.lax.dot_general(wb, fb, (((1,), (0,)), ((), ())),
                                preferred_element_type=jnp.float32)
        st_ref[0, :] += jnp.sum(o, axis=1)
        st_ref[1, :] += jnp.sum(o * o, axis=1)
        u_ref[0] = jnp.transpose(o, (1, 0))

    return pl.pallas_call(
        body,
        grid=(B, ns),
        in_specs=[
            pl.BlockSpec((1, C, strip), lambda b, i: (b, 0, i)),
            pl.BlockSpec((64, C), lambda b, i: (0, 0)),
        ],
        out_specs=[
            pl.BlockSpec((1, strip, 64), lambda b, i: (b, i, 0)),
            pl.BlockSpec((2, 64), lambda b, i: (0, 0)),
        ],
        out_shape=[
            jax.ShapeDtypeStruct((B, H * W, 64), jnp.float32),
            jax.ShapeDtypeStruct((2, 64), jnp.float32),
        ],
    )


@functools.cache
def _make_fr2(H, W, B):
    srows, strip, ns = _strip_cfg(H, W)
    inv_n = 1.0 / (B * H * W)

    def body(x_ref, up_ref, dn_ref, st_u_ref, g_ref, bb_ref, w_ref, v_ref, st_ref):
        b = pl.program_id(0)
        i = pl.program_id(1)

        @pl.when(jnp.logical_and(b == 0, i == 0))
        def _():
            st_ref[...] = jnp.zeros_like(st_ref)

        mean = st_u_ref[0:1, :] * inv_n
        var = st_u_ref[1:2, :] * inv_n - mean * mean
        scale = g_ref[...] / jnp.sqrt(var + _BN_EPS)
        shift = bb_ref[...] - mean * scale

        def r1(x):
            return jax.nn.relu(x * scale + shift).astype(jnp.bfloat16)

        above, below = _edge_halo(r1(up_ref[0]), r1(dn_ref[0]), i, ns)
        xb = jnp.concatenate([above, r1(x_ref[0]), below], axis=0)
        o = _conv3x3_taps([xb], [w_ref[...].astype(jnp.bfloat16)], strip, W, _CH)
        v_ref[0] = o
        st_ref[0, :] += jnp.sum(o, axis=0)
        st_ref[1, :] += jnp.sum(o * o, axis=0)

    return pl.pallas_call(
        body,
        grid=(B, ns),
        in_specs=_halo_specs(H, W, 64, srows, strip) + [
            pl.BlockSpec((2, 64), lambda b, i: (0, 0)),
            pl.BlockSpec((1, 64), lambda b, i: (0, 0)),
            pl.BlockSpec((1, 64), lambda b, i: (0, 0)),
            pl.BlockSpec((3, 3, 64, _CH), lambda b, i: (0, 0, 0, 0)),
        ],
        out_specs=[
            pl.BlockSpec((1, strip, _CH), lambda b, i: (b, i, 0)),
            pl.BlockSpec((2, _CH), lambda b, i: (0, 0)),
        ],
        out_shape=[
            jax.ShapeDtypeStruct((B, H * W, _CH), jnp.float32),
            jax.ShapeDtypeStruct((2, _CH), jnp.float32),
        ],
    )


@functools.cache
def _make_cn1(H, W, B):
    srows, strip, ns = _strip_cfg(H, W)
    inv_n = 1.0 / (B * H * W)

    def body(dl_ref, dlu_ref, dld_ref, v_ref, vu_ref, vd_ref,
             st_v_ref, g_ref, bb_ref, wdl_ref, wrd_ref, o_ref, st_ref):
        b = pl.program_id(0)
        i = pl.program_id(1)

        @pl.when(jnp.logical_and(b == 0, i == 0))
        def _():
            st_ref[...] = jnp.zeros_like(st_ref)

        mean = st_v_ref[0:1, :] * inv_n
        var = st_v_ref[1:2, :] * inv_n - mean * mean
        scale = g_ref[...] / jnp.sqrt(var + _BN_EPS)
        shift = bb_ref[...] - mean * scale

        def red(x):
            return jax.nn.relu(x * scale + shift).astype(jnp.bfloat16)

        vau, vbd = _edge_halo(red(vu_ref[0]), red(vd_ref[0]), i, ns)
        vb = jnp.concatenate([vau, red(v_ref[0]), vbd], axis=0)
        dau, dbd = _edge_halo(dlu_ref[0].astype(jnp.bfloat16),
                              dld_ref[0].astype(jnp.bfloat16), i, ns)
        db = jnp.concatenate([dau, dl_ref[0].astype(jnp.bfloat16), dbd], axis=0)

        o = _conv3x3_taps([db, vb],
                          [wdl_ref[...].astype(jnp.bfloat16),
                           wrd_ref[...].astype(jnp.bfloat16)], strip, W, 64)
        o_ref[0] = o
        st_ref[0, :] += jnp.sum(o, axis=0)
        st_ref[1, :] += jnp.sum(o * o, axis=0)

    return pl.pallas_call(
        body,
        grid=(B, ns),
        in_specs=(_halo_specs(H, W, 64, srows, strip)
                  + _halo_specs(H, W, _CH, srows, strip)
                  + [
                      pl.BlockSpec((2, _CH), lambda b, i: (0, 0)),
                      pl.BlockSpec((1, _CH), lambda b, i: (0, 0)),
                      pl.BlockSpec((1, _CH), lambda b, i: (0, 0)),
                      pl.BlockSpec((3, 3, 64, 64), lambda b, i: (0, 0, 0, 0)),
                      pl.BlockSpec((3, 3, _CH, 64), lambda b, i: (0, 0, 0, 0)),
                  ]),
        out_specs=[
            pl.BlockSpec((1, strip, 64), lambda b, i: (b, i, 0)),
            pl.BlockSpec((2, 64), lambda b, i: (0, 0)),
        ],
        out_shape=[
            jax.ShapeDtypeStruct((B, H * W, 64), jnp.float32),
            jax.ShapeDtypeStruct((2, 64), jnp.float32),
        ],
    )


@functools.cache
def _make_rows(H, W, B):
    srows, strip, ns = _strip_cfg(H, W)
    inv_n = 1.0 / (B * H * W)

    def body(wp_ref, st_w_ref, cg_ref, cb_ref, w2_ref, b2_ref,
             v_ref, st_v_ref, g_ref, bb_ref, valid_ref, o_ref):
        meanw = st_w_ref[0:1, :] * inv_n
        varw = st_w_ref[1:2, :] * inv_n - meanw * meanw
        scw = cg_ref[...] / jnp.sqrt(varw + _BN_EPS)
        shw = cb_ref[...] - meanw * scw
        c = jax.nn.relu(wp_ref[0] * scw + shw).astype(jnp.bfloat16)
        w2 = w2_ref[...].astype(jnp.bfloat16).astype(jnp.float32)
        logit = jnp.sum(c.astype(jnp.float32) * w2, axis=1, keepdims=True)
        conf = jax.nn.sigmoid(logit + b2_ref[...])

        meanv = st_v_ref[0:1, :] * inv_n
        varv = st_v_ref[1:2, :] * inv_n - meanv * meanv
        scv = g_ref[...] / jnp.sqrt(varv + _BN_EPS)
        shv = bb_ref[...] - meanv * scv
        reduced = jax.nn.relu(v_ref[0] * scv + shv)
        o_ref[0] = reduced * conf * valid_ref[0, :, 0:1]

    return pl.pallas_call(
        body,
        grid=(B, ns),
        in_specs=[
            pl.BlockSpec((1, strip, 64), lambda b, i: (b, i, 0)),
            pl.BlockSpec((2, 64), lambda b, i: (0, 0)),
            pl.BlockSpec((1, 64), lambda b, i: (0, 0)),
            pl.BlockSpec((1, 64), lambda b, i: (0, 0)),
            pl.BlockSpec((1, 64), lambda b, i: (0, 0)),
            pl.BlockSpec((1, 1), lambda b, i: (0, 0)),
            pl.BlockSpec((1, strip, _CH), lambda b, i: (b, i, 0)),
            pl.BlockSpec((2, _CH), lambda b, i: (0, 0)),
            pl.BlockSpec((1, _CH), lambda b, i: (0, 0)),
            pl.BlockSpec((1, _CH), lambda b, i: (0, 0)),
            pl.BlockSpec((1, strip, 1), lambda b, i: (b, i, 0)),
        ],
        out_specs=pl.BlockSpec((1, strip, _CH), lambda b, i: (b, i, 0)),
        out_shape=jax.ShapeDtypeStruct((B, H * W, _CH), jnp.float32),
    )


_BN_EPS = 1e-5


_NSTRIP = 8
_STRIP = _NCELL // _NSTRIP
_SROWS = _STRIP // _BEV_W


@functools.cache
def _make_skip_head(co, B):
    inv_n = 1.0 / (B * _NCELL)

    def body(x_ref, w_ref, g_ref, b_ref, o_ref, m_sc, s_sc):
        p = pl.program_id(0)
        b = pl.program_id(1)
        i = pl.program_id(2)

        @pl.when(jnp.logical_and(p == 0, jnp.logical_and(b == 0, i == 0)))
        def _():
            m_sc[...] = jnp.zeros_like(m_sc)
            s_sc[...] = jnp.zeros_like(s_sc)

        xb = x_ref[0].astype(jnp.bfloat16)
        wb = w_ref[...].astype(jnp.bfloat16)
        o = jax.lax.dot_general(xb, wb, (((1,), (1,)), ((), ())),
                                preferred_element_type=jnp.float32)

        @pl.when(p == 0)
        def _():
            m_sc[...] += jnp.sum(o, axis=0, keepdims=True)
            s_sc[...] += jnp.sum(o * o, axis=0, keepdims=True)

        @pl.when(p == 1)
        def _():
            mean = m_sc[...] * inv_n
            var = s_sc[...] * inv_n - mean * mean
            scale = g_ref[...] / jnp.sqrt(var + _BN_EPS)
            shift = b_ref[...] - mean * scale
            y = jax.nn.relu(o * scale + shift)
            o_ref[0] = jnp.transpose(y, (1, 0)).reshape(co, _SROWS, _BEV_W)

    return pl.pallas_call(
        body,
        grid=(2, B, _NSTRIP),
        in_specs=[
            pl.BlockSpec((1, _STRIP, _CH), lambda p, b, i: (b, i, 0)),
            pl.BlockSpec((co, _CH), lambda p, b, i: (0, 0)),
            pl.BlockSpec((1, co), lambda p, b, i: (0, 0)),
            pl.BlockSpec((1, co), lambda p, b, i: (0, 0)),
        ],
        out_specs=pl.BlockSpec((1, co, _SROWS, _BEV_W), lambda p, b, i: (b, 0, i, 0)),
        out_shape=jax.ShapeDtypeStruct((B, co, _BEV_H, _BEV_W), jnp.float32),
        scratch_shapes=[
            pltpu.VMEM((1, co), jnp.float32),
            pltpu.VMEM((1, co), jnp.float32),
        ],
    )


@functools.cache
def _make_main_head(B):
    inv_n = 1.0 / (B * _NCELL)

    def body(x_ref, up_ref, dn_ref, w_ref, g_ref, b_ref, o_ref, m_sc, s_sc):
        p = pl.program_id(0)
        b = pl.program_id(1)
        i = pl.program_id(2)

        @pl.when(jnp.logical_and(p == 0, jnp.logical_and(b == 0, i == 0)))
        def _():
            m_sc[...] = jnp.zeros_like(m_sc)
            s_sc[...] = jnp.zeros_like(s_sc)

        above = up_ref[0].astype(jnp.bfloat16) * (i > 0).astype(jnp.bfloat16)
        below = dn_ref[0].astype(jnp.bfloat16) * (i < _NSTRIP - 1).astype(jnp.bfloat16)
        xb = jnp.concatenate([above, x_ref[0].astype(jnp.bfloat16), below], axis=0)

        wv = w_ref[...].astype(jnp.bfloat16)
        wq = jax.lax.broadcasted_iota(jnp.int32, (_STRIP, 1), 0) % _BEV_W
        o = jnp.zeros((_STRIP, _CH), jnp.float32)
        for kx in range(3):
            a = jnp.zeros((_STRIP, _CH), jnp.float32)
            for ky in range(3):
                src = jax.lax.slice(xb, (ky * _BEV_W, 0), (ky * _BEV_W + _STRIP, _CH))
                a = a + jax.lax.dot_general(src, wv[ky, kx], (((1,), (0,)), ((), ())),
                                            preferred_element_type=jnp.float32)
            if kx == 0:
                sh = jnp.concatenate([jnp.zeros((1, _CH), jnp.float32), a[:-1]], axis=0)
                o = o + sh * (wq >= 1).astype(jnp.float32)
            elif kx == 1:
                o = o + a
            else:
                sh = jnp.concatenate([a[1:], jnp.zeros((1, _CH), jnp.float32)], axis=0)
                o = o + sh * (wq <= _BEV_W - 2).astype(jnp.float32)

        @pl.when(p == 0)
        def _():
            m_sc[...] += jnp.sum(o, axis=0, keepdims=True)
            s_sc[...] += jnp.sum(o * o, axis=0, keepdims=True)

        @pl.when(p == 1)
        def _():
            mean = m_sc[...] * inv_n
            var = s_sc[...] * inv_n - mean * mean
            scale = g_ref[...] / jnp.sqrt(var + _BN_EPS)
            shift = b_ref[...] - mean * scale
            y = jax.nn.relu(o * scale + shift)
            o_ref[0] = jnp.transpose(y, (1, 0)).reshape(_CH, _SROWS, _BEV_W)

    nrow = _NCELL // _BEV_W
    return pl.pallas_call(
        body,
        grid=(2, B, _NSTRIP),
        in_specs=[
            pl.BlockSpec((1, _STRIP, _CH), lambda p, b, i: (b, i, 0)),
            pl.BlockSpec((1, _BEV_W, _CH),
                         lambda p, b, i: (b, jnp.maximum(i * _SROWS - 1, 0), 0)),
            pl.BlockSpec((1, _BEV_W, _CH),
                         lambda p, b, i: (b, jnp.minimum((i + 1) * _SROWS, nrow - 1), 0)),
            pl.BlockSpec((3, 3, _CH, _CH), lambda p, b, i: (0, 0, 0, 0)),
            pl.BlockSpec((1, _CH), lambda p, b, i: (0, 0)),
            pl.BlockSpec((1, _CH), lambda p, b, i: (0, 0)),
        ],
        out_specs=pl.BlockSpec((1, _CH, _SROWS, _BEV_W), lambda p, b, i: (b, 0, i, 0)),
        out_shape=jax.ShapeDtypeStruct((B, _CH, _BEV_H, _BEV_W), jnp.float32),
        scratch_shapes=[
            pltpu.VMEM((1, _CH), jnp.float32),
            pltpu.VMEM((1, _CH), jnp.float32),
        ],
    )


def _conv2d(x, w, bias=None, padding=0, groups=1):
    out = lax.conv_general_dilated(
        x, w, window_strides=(1, 1),
        padding=[(padding, padding), (padding, padding)],
        feature_group_count=groups,
        dimension_numbers=("NCHW", "OIHW", "NCHW"))
    if bias is not None:
        out = out + bias[None, :, None, None]
    return out


def _bnorm(x, g, b, eps=1e-5):
    m = jnp.mean(x, axis=(0, 2, 3), keepdims=True)
    v = jnp.var(x, axis=(0, 2, 3), keepdims=True)
    return (x - m) / jnp.sqrt(v + eps) * g[None, :, None, None] + b[None, :, None, None]


def _pixel_grid(H, W):
    x = jnp.linspace(0.0, W - 1.0, W)
    y = jnp.linspace(0.0, H - 1.0, H)
    yy, xx = jnp.meshgrid(y, x, indexing="ij")
    return jnp.stack([xx, yy, jnp.ones_like(xx)], axis=-1).reshape(-1, 3).T


def kernel(feat_stage1, feat_stage2, feat_stage3, feat_stage5, intrinsics, extrinsics, params):
    feats = {"stage1": feat_stage1, "stage2": feat_stage2, "stage3": feat_stage3, "stage5": feat_stage5}
    Bsz = feat_stage1.shape[0]
    depth_bins = jnp.exp(jnp.linspace(jnp.log(1.0), jnp.log(60.0), _DEPTH_CH))
    K_inv = jnp.linalg.inv(intrinsics)

    accs = {}
    for s in _SCALES:
        p = params[s]
        f = feats[s]
        H, W = f.shape[2], f.shape[3]
        N = H * W

        h = jax.nn.relu(_bnorm(_conv2d(f, p["dn_w1"]), p["dn_g1"], p["dn_b1"]))
        depth_logits = _conv2d(h, p["dn_w2"], bias=p["dn_bias2"])
        depth_probs = jax.nn.softmax(depth_logits * 10.0, axis=1)
        depth_map = jnp.sum(depth_probs * depth_bins[None, :, None, None], axis=1)
        pixels = jnp.broadcast_to(_pixel_grid(H, W)[None], (Bsz, 3, H * W))
        dm = depth_map.reshape(Bsz, 1, -1)
        cam_pts = dm * jnp.einsum("bij,bjn->bin", K_inv, pixels)
        cam_pts_h = jnp.concatenate([cam_pts, jnp.ones_like(cam_pts[:, :1])], axis=1)
        ego = jnp.einsum("bij,bjn->bin", extrinsics, cam_pts_h)[:, :3]
        bev_x = (ego[:, 0] / _VOX[0] + _BEV_W // 2).astype(jnp.int32)
        bev_y = (ego[:, 1] / _VOX[1] + _BEV_H // 2).astype(jnp.int32)
        valid = (bev_x >= 0) & (bev_x < _BEV_W) & (bev_y >= 0) & (bev_y < _BEV_H)
        idx = jnp.where(valid, bev_y * _BEV_W + bev_x, 0)

        C = f.shape[1]
        u1, st_u = _make_fr1(C, H, W, Bsz)(
            f.reshape(Bsz, C, N), p["fr_w1"].reshape(64, C))
        w2d = jnp.zeros((_CH, 64, 3, 3), jnp.float32)
        for g in range(8):
            w2d = w2d.at[g * 16:(g + 1) * 16, g * 8:(g + 1) * 8].set(
                p["fr_w2"][g * 16:(g + 1) * 16])
        w2d = jnp.transpose(w2d, (2, 3, 1, 0))
        v, st_v = _make_fr2(H, W, Bsz)(
            u1, u1, u1, st_u, p["fr_g1"].reshape(1, 64), p["fr_b1"].reshape(1, 64), w2d)

        dlp = jnp.transpose(depth_logits.reshape(Bsz, 64, N), (0, 2, 1))
        wdl = jnp.transpose(p["cn_w1"][:, :64], (2, 3, 1, 0))
        wrd = jnp.transpose(p["cn_w1"][:, 64:], (2, 3, 1, 0))
        g2 = p["fr_g2"].reshape(1, _CH)
        b2 = p["fr_b2"].reshape(1, _CH)
        wp, st_w = _make_cn1(H, W, Bsz)(
            dlp, dlp, dlp, v, v, v, st_v, g2, b2, wdl, wrd)

        rows = _make_rows(H, W, Bsz)(
            wp, st_w, p["cn_g1"].reshape(1, 64), p["cn_b1"].reshape(1, 64),
            p["cn_w2"].reshape(1, 64), p["cn_bias2"].reshape(1, 1),
            v, st_v, g2, b2,
            valid.astype(jnp.float32).reshape(Bsz, N, 1))
        accs[s] = _make_sc_scatter(H * W, Bsz)(idx.astype(jnp.int32), rows)

    outs = []
    for s in _SKIP:
        sp = params["sp_" + s]
        co = sp["w"].shape[0]
        outs.append(_make_skip_head(co, Bsz)(
            accs[s], sp["w"].reshape(co, _CH),
            sp["g"].reshape(1, co), sp["b"].reshape(1, co)))
    mp = params["mp"]
    main = _make_main_head(Bsz)(
        accs["stage5"], accs["stage5"], accs["stage5"],
        jnp.transpose(mp["w"], (2, 3, 1, 0)),
        mp["g"].reshape(1, _CH), mp["b"].reshape(1, _CH))
    return (main, outs[0], outs[1], outs[2])


# speedup vs baseline: 15.2749x; 1.0777x over previous
import functools

import jax
import jax.numpy as jnp
from jax import lax
from jax.experimental import pallas as pl
from jax.experimental.pallas import tpu as pltpu
from jax.experimental.pallas import tpu_sc as plsc

_SCALES = ["stage1", "stage2", "stage3", "stage5"]
_HWS = {"stage1": (128, 128), "stage2": (64, 64), "stage3": (32, 32), "stage5": (16, 16)}
_SKIP = ["stage1", "stage2", "stage3"]
_DEPTH_CH = 64
_BEV_H = 128
_BEV_W = 128
_NCELL = _BEV_H * _BEV_W
_VOX = (0.4, 0.4)

_CH = 128
_HALF = _NCELL // 2
_DUMP = _HALF
_ACC_ROWS = _HALF + 128
_ZROWS = 130


@functools.cache
def _make_sc_scatter(N, B):
    mesh = plsc.VectorSubcoreMesh(core_axis_name="c", subcore_axis_name="s")
    chunk = N // 16
    K = min(128, chunk)
    nk = chunk // K

    @functools.partial(
        pl.kernel,
        mesh=mesh,
        out_type=jax.ShapeDtypeStruct((B, _NCELL, _CH), jnp.float32),
        scratch_types=[
            pltpu.VMEM_SHARED((_ACC_ROWS, _CH), jnp.float32),
            pltpu.VMEM((K,), jnp.int32),
            pltpu.VMEM((K,), jnp.int32),
            pltpu.VMEM((K, _CH), jnp.float32),
            pltpu.VMEM((K, _CH), jnp.float32),
            pltpu.VMEM((_ZROWS, _CH), jnp.float32),
            pltpu.SemaphoreType.DMA,
            pltpu.SemaphoreType.DMA,
            pltpu.SemaphoreType.DMA,
            pltpu.SemaphoreType.DMA,
            pltpu.SemaphoreType.DMA,
        ],
    )
    def kfn(idx_hbm, rows_hbm, acc_hbm, acc_sh, idx0, idx1, rows0, rows1,
            zbuf, sl0, sl1, sc0, sc1, sw):
        c = lax.axis_index("c")
        s = lax.axis_index("s")
        half_base = c * _HALF
        idx_v = [idx0, idx1]
        rows_v = [rows0, rows1]
        sem_ld = [sl0, sl1]
        sem_sc = [sc0, sc1]

        def zrow(i, _):
            def zcol(j, __):
                zbuf[i, pl.ds(j * 16, 16)] = jnp.zeros((16,), jnp.float32)
                return 0
            return lax.fori_loop(0, _CH // 16, zcol, 0)
        lax.fori_loop(0, _ZROWS, zrow, 0)

        def load(b, t, p):
            off = s * chunk + t * K
            ha = pltpu.async_copy(idx_hbm.at[b, pl.ds(off, K)], idx_v[p], sem_ld[p])
            hb = pltpu.async_copy(rows_hbm.at[b, pl.ds(off, K)], rows_v[p], sem_ld[p])
            return ha, hb

        for b in range(B):
            hz = [pltpu.async_copy(
                      zbuf, acc_sh.at[pl.ds(s * 4 * _ZROWS + j * _ZROWS, _ZROWS)], sw)
                  for j in range(4)]
            for h in hz:
                h.wait()
            plsc.subcore_barrier()

            pend = load(b, 0, 0)
            hsc = [None, None]
            for t in range(nk):
                p = t % 2
                for h in pend:
                    h.wait()
                for v in range(K // 16):
                    t16 = idx_v[p][pl.ds(v * 16, 16)]
                    loc = t16 - half_base
                    ok = (loc >= 0) & (loc < _HALF)
                    idx_v[p][pl.ds(v * 16, 16)] = jnp.where(ok, loc, _DUMP)
                hsc[p] = pltpu.async_copy(rows_v[p], acc_sh.at[idx_v[p]],
                                          sem_sc[p], add=True)
                if t + 1 < nk:
                    if hsc[1 - p] is not None:
                        hsc[1 - p].wait()
                        hsc[1 - p] = None
                    pend = load(b, t + 1, 1 - p)
            for p in range(2):
                if hsc[p] is not None:
                    hsc[p].wait()
            plsc.subcore_barrier()

            hw = [pltpu.async_copy(
                      acc_sh.at[pl.ds(s * 512 + j * 128, 128)],
                      acc_hbm.at[b, pl.ds(half_base + s * 512 + j * 128, 128)], sw)
                  for j in range(4)]
            for h in hw:
                h.wait()
            plsc.subcore_barrier()

    return kfn


def _strip_cfg(H, W):
    srows = min(16, H)
    return srows, srows * W, H // srows


def _conv3x3_taps(xbs, wv, strip, W, CO):
    wq = jax.lax.broadcasted_iota(jnp.int32, (strip, 1), 0) % W
    o = jnp.zeros((strip, CO), jnp.float32)
    for kx in range(3):
        a = jnp.zeros((strip, CO), jnp.float32)
        for ky in range(3):
            for xb, w in zip(xbs, wv):
                src = jax.lax.slice(xb, (ky * W, 0), (ky * W + strip, xb.shape[1]))
                a = a + jax.lax.dot_general(src, w[ky, kx], (((1,), (0,)), ((), ())),
                                            preferred_element_type=jnp.float32)
        if kx == 0:
            sh = jnp.concatenate([jnp.zeros((1, CO), jnp.float32), a[:-1]], axis=0)
            o = o + sh * (wq >= 1).astype(jnp.float32)
        elif kx == 1:
            o = o + a
        else:
            sh = jnp.concatenate([a[1:], jnp.zeros((1, CO), jnp.float32)], axis=0)
            o = o + sh * (wq <= W - 2).astype(jnp.float32)
    return o


def _halo_specs(H, W, CI, srows, strip):
    return [
        pl.BlockSpec((1, strip, CI), lambda b, i: (b, i, 0)),
        pl.BlockSpec((1, W, CI), lambda b, i: (b, jnp.maximum(i * srows - 1, 0), 0)),
        pl.BlockSpec((1, W, CI), lambda b, i: (b, jnp.minimum((i + 1) * srows, H - 1), 0)),
    ]


def _edge_halo(up, dn, i, ns):
    above = up * (i > 0).astype(up.dtype)
    below = dn * (i < ns - 1).astype(dn.dtype)
    return above, below


@functools.cache
def _make_fr1(C, H, W, B):
    srows, strip, ns = _strip_cfg(H, W)

    def body(f_ref, w_ref, u_ref, st_ref):
        b = pl.program_id(0)
        i = pl.program_id(1)

        @pl.when(jnp.logical_and(b == 0, i == 0))
        def _():
            st_ref[...] = jnp.zeros_like(st_ref)

        fb = f_ref[0].astype(jnp.bfloat16)
        wb = w_ref[...].astype(jnp.bfloat16)
        o = jax.lax.dot_general(wb, fb, (((1,), (0,)), ((), ())),
                                preferred_element_type=jnp.float32)
        st_ref[0, :] += jnp.sum(o, axis=1)
        st_ref[1, :] += jnp.sum(o * o, axis=1)
        u_ref[0] = jnp.transpose(o, (1, 0))

    return pl.pallas_call(
        body,
        grid=(B, ns),
        in_specs=[
            pl.BlockSpec((1, C, strip), lambda b, i: (b, 0, i)),
            pl.BlockSpec((64, C), lambda b, i: (0, 0)),
        ],
        out_specs=[
            pl.BlockSpec((1, strip, 64), lambda b, i: (b, i, 0)),
            pl.BlockSpec((2, 64), lambda b, i: (0, 0)),
        ],
        out_shape=[
            jax.ShapeDtypeStruct((B, H * W, 64), jnp.float32),
            jax.ShapeDtypeStruct((2, 64), jnp.float32),
        ],
    )


@functools.cache
def _make_fr2(H, W, B):
    srows, strip, ns = _strip_cfg(H, W)
    inv_n = 1.0 / (B * H * W)

    def body(x_ref, up_ref, dn_ref, st_u_ref, g_ref, bb_ref, w_ref, v_ref, st_ref):
        b = pl.program_id(0)
        i = pl.program_id(1)

        @pl.when(jnp.logical_and(b == 0, i == 0))
        def _():
            st_ref[...] = jnp.zeros_like(st_ref)

        mean = st_u_ref[0:1, :] * inv_n
        var = st_u_ref[1:2, :] * inv_n - mean * mean
        scale = g_ref[...] / jnp.sqrt(var + _BN_EPS)
        shift = bb_ref[...] - mean * scale

        def r1(x):
            return jax.nn.relu(x * scale + shift).astype(jnp.bfloat16)

        above, below = _edge_halo(r1(up_ref[0]), r1(dn_ref[0]), i, ns)
        xb = jnp.concatenate([above, r1(x_ref[0]), below], axis=0)
        o = _conv3x3_taps([xb], [w_ref[...].astype(jnp.bfloat16)], strip, W, _CH)
        v_ref[0] = o
        st_ref[0, :] += jnp.sum(o, axis=0)
        st_ref[1, :] += jnp.sum(o * o, axis=0)

    return pl.pallas_call(
        body,
        grid=(B, ns),
        in_specs=_halo_specs(H, W, 64, srows, strip) + [
            pl.BlockSpec((2, 64), lambda b, i: (0, 0)),
            pl.BlockSpec((1, 64), lambda b, i: (0, 0)),
            pl.BlockSpec((1, 64), lambda b, i: (0, 0)),
            pl.BlockSpec((3, 3, 64, _CH), lambda b, i: (0, 0, 0, 0)),
        ],
        out_specs=[
            pl.BlockSpec((1, strip, _CH), lambda b, i: (b, i, 0)),
            pl.BlockSpec((2, _CH), lambda b, i: (0, 0)),
        ],
        out_shape=[
            jax.ShapeDtypeStruct((B, H * W, _CH), jnp.float32),
            jax.ShapeDtypeStruct((2, _CH), jnp.float32),
        ],
    )


@functools.cache
def _make_cn1(H, W, B):
    srows, strip, ns = _strip_cfg(H, W)
    inv_n = 1.0 / (B * H * W)

    def body(dl_ref, dlu_ref, dld_ref, v_ref, vu_ref, vd_ref,
             st_v_ref, g_ref, bb_ref, wdl_ref, wrd_ref, o_ref, st_ref):
        b = pl.program_id(0)
        i = pl.program_id(1)

        @pl.when(jnp.logical_and(b == 0, i == 0))
        def _():
            st_ref[...] = jnp.zeros_like(st_ref)

        mean = st_v_ref[0:1, :] * inv_n
        var = st_v_ref[1:2, :] * inv_n - mean * mean
        scale = g_ref[...] / jnp.sqrt(var + _BN_EPS)
        shift = bb_ref[...] - mean * scale

        def red(x):
            return jax.nn.relu(x * scale + shift).astype(jnp.bfloat16)

        vau, vbd = _edge_halo(red(vu_ref[0]), red(vd_ref[0]), i, ns)
        vb = jnp.concatenate([vau, red(v_ref[0]), vbd], axis=0)
        dau, dbd = _edge_halo(dlu_ref[0].astype(jnp.bfloat16),
                              dld_ref[0].astype(jnp.bfloat16), i, ns)
        db = jnp.concatenate([dau, dl_ref[0].astype(jnp.bfloat16), dbd], axis=0)

        o = _conv3x3_taps([db, vb],
                          [wdl_ref[...].astype(jnp.bfloat16),
                           wrd_ref[...].astype(jnp.bfloat16)], strip, W, 64)
        o_ref[0] = o
        st_ref[0, :] += jnp.sum(o, axis=0)
        st_ref[1, :] += jnp.sum(o * o, axis=0)

    return pl.pallas_call(
        body,
        grid=(B, ns),
        in_specs=(_halo_specs(H, W, 64, srows, strip)
                  + _halo_specs(H, W, _CH, srows, strip)
                  + [
                      pl.BlockSpec((2, _CH), lambda b, i: (0, 0)),
                      pl.BlockSpec((1, _CH), lambda b, i: (0, 0)),
                      pl.BlockSpec((1, _CH), lambda b, i: (0, 0)),
                      pl.BlockSpec((3, 3, 64, 64), lambda b, i: (0, 0, 0, 0)),
                      pl.BlockSpec((3, 3, _CH, 64), lambda b, i: (0, 0, 0, 0)),
                  ]),
        out_specs=[
            pl.BlockSpec((1, strip, 64), lambda b, i: (b, i, 0)),
            pl.BlockSpec((2, 64), lambda b, i: (0, 0)),
        ],
        out_shape=[
            jax.ShapeDtypeStruct((B, H * W, 64), jnp.float32),
            jax.ShapeDtypeStruct((2, 64), jnp.float32),
        ],
    )


@functools.cache
def _make_rows(H, W, B):
    srows, strip, ns = _strip_cfg(H, W)
    inv_n = 1.0 / (B * H * W)

    def body(wp_ref, st_w_ref, cg_ref, cb_ref, w2_ref, b2_ref,
             v_ref, st_v_ref, g_ref, bb_ref, valid_ref, o_ref):
        meanw = st_w_ref[0:1, :] * inv_n
        varw = st_w_ref[1:2, :] * inv_n - meanw * meanw
        scw = cg_ref[...] / jnp.sqrt(varw + _BN_EPS)
        shw = cb_ref[...] - meanw * scw
        c = jax.nn.relu(wp_ref[0] * scw + shw).astype(jnp.bfloat16)
        w2 = w2_ref[...].astype(jnp.bfloat16).astype(jnp.float32)
        logit = jnp.sum(c.astype(jnp.float32) * w2, axis=1, keepdims=True)
        conf = jax.nn.sigmoid(logit + b2_ref[...])

        meanv = st_v_ref[0:1, :] * inv_n
        varv = st_v_ref[1:2, :] * inv_n - meanv * meanv
        scv = g_ref[...] / jnp.sqrt(varv + _BN_EPS)
        shv = bb_ref[...] - meanv * scv
        reduced = jax.nn.relu(v_ref[0] * scv + shv)
        o_ref[0] = reduced * conf * valid_ref[0, :, 0:1]

    return pl.pallas_call(
        body,
        grid=(B, ns),
        in_specs=[
            pl.BlockSpec((1, strip, 64), lambda b, i: (b, i, 0)),
            pl.BlockSpec((2, 64), lambda b, i: (0, 0)),
            pl.BlockSpec((1, 64), lambda b, i: (0, 0)),
            pl.BlockSpec((1, 64), lambda b, i: (0, 0)),
            pl.BlockSpec((1, 64), lambda b, i: (0, 0)),
            pl.BlockSpec((1, 1), lambda b, i: (0, 0)),
            pl.BlockSpec((1, strip, _CH), lambda b, i: (b, i, 0)),
            pl.BlockSpec((2, _CH), lambda b, i: (0, 0)),
            pl.BlockSpec((1, _CH), lambda b, i: (0, 0)),
            pl.BlockSpec((1, _CH), lambda b, i: (0, 0)),
            pl.BlockSpec((1, strip, 1), lambda b, i: (b, i, 0)),
        ],
        out_specs=pl.BlockSpec((1, strip, _CH), lambda b, i: (b, i, 0)),
        out_shape=jax.ShapeDtypeStruct((B, H * W, _CH), jnp.float32),
    )


_BN_EPS = 1e-5


_NSTRIP = 8
_STRIP = _NCELL // _NSTRIP
_SROWS = _STRIP // _BEV_W


@functools.cache
def _make_skip_head(co, B):
    inv_n = 1.0 / (B * _NCELL)

    def body(x_ref, w_ref, g_ref, b_ref, o_ref, m_sc, s_sc):
        p = pl.program_id(0)
        b = pl.program_id(1)
        i = pl.program_id(2)

        @pl.when(jnp.logical_and(p == 0, jnp.logical_and(b == 0, i == 0)))
        def _():
            m_sc[...] = jnp.zeros_like(m_sc)
            s_sc[...] = jnp.zeros_like(s_sc)

        xb = x_ref[0].astype(jnp.bfloat16)
        wb = w_ref[...].astype(jnp.bfloat16)
        o = jax.lax.dot_general(xb, wb, (((1,), (1,)), ((), ())),
                                preferred_element_type=jnp.float32)

        @pl.when(p == 0)
        def _():
            m_sc[...] += jnp.sum(o, axis=0, keepdims=True)
            s_sc[...] += jnp.sum(o * o, axis=0, keepdims=True)

        @pl.when(p == 1)
        def _():
            mean = m_sc[...] * inv_n
            var = s_sc[...] * inv_n - mean * mean
            scale = g_ref[...] / jnp.sqrt(var + _BN_EPS)
            shift = b_ref[...] - mean * scale
            y = jax.nn.relu(o * scale + shift)
            o_ref[0] = jnp.transpose(y, (1, 0)).reshape(co, _SROWS, _BEV_W)

    return pl.pallas_call(
        body,
        grid=(2, B, _NSTRIP),
        in_specs=[
            pl.BlockSpec((1, _STRIP, _CH), lambda p, b, i: (b, i, 0)),
            pl.BlockSpec((co, _CH), lambda p, b, i: (0, 0)),
            pl.BlockSpec((1, co), lambda p, b, i: (0, 0)),
            pl.BlockSpec((1, co), lambda p, b, i: (0, 0)),
        ],
        out_specs=pl.BlockSpec((1, co, _SROWS, _BEV_W), lambda p, b, i: (b, 0, i, 0)),
        out_shape=jax.ShapeDtypeStruct((B, co, _BEV_H, _BEV_W), jnp.float32),
        scratch_shapes=[
            pltpu.VMEM((1, co), jnp.float32),
            pltpu.VMEM((1, co), jnp.float32),
        ],
    )


@functools.cache
def _make_main_head(B):
    inv_n = 1.0 / (B * _NCELL)

    def body(x_ref, up_ref, dn_ref, w_ref, g_ref, b_ref, o_ref, m_sc, s_sc):
        p = pl.program_id(0)
        b = pl.program_id(1)
        i = pl.program_id(2)

        @pl.when(jnp.logical_and(p == 0, jnp.logical_and(b == 0, i == 0)))
        def _():
            m_sc[...] = jnp.zeros_like(m_sc)
            s_sc[...] = jnp.zeros_like(s_sc)

        above = up_ref[0].astype(jnp.bfloat16) * (i > 0).astype(jnp.bfloat16)
        below = dn_ref[0].astype(jnp.bfloat16) * (i < _NSTRIP - 1).astype(jnp.bfloat16)
        xb = jnp.concatenate([above, x_ref[0].astype(jnp.bfloat16), below], axis=0)

        wv = w_ref[...].astype(jnp.bfloat16)
        wq = jax.lax.broadcasted_iota(jnp.int32, (_STRIP, 1), 0) % _BEV_W
        o = jnp.zeros((_STRIP, _CH), jnp.float32)
        for kx in range(3):
            a = jnp.zeros((_STRIP, _CH), jnp.float32)
            for ky in range(3):
                src = jax.lax.slice(xb, (ky * _BEV_W, 0), (ky * _BEV_W + _STRIP, _CH))
                a = a + jax.lax.dot_general(src, wv[ky, kx], (((1,), (0,)), ((), ())),
                                            preferred_element_type=jnp.float32)
            if kx == 0:
                sh = jnp.concatenate([jnp.zeros((1, _CH), jnp.float32), a[:-1]], axis=0)
                o = o + sh * (wq >= 1).astype(jnp.float32)
            elif kx == 1:
                o = o + a
            else:
                sh = jnp.concatenate([a[1:], jnp.zeros((1, _CH), jnp.float32)], axis=0)
                o = o + sh * (wq <= _BEV_W - 2).astype(jnp.float32)

        @pl.when(p == 0)
        def _():
            m_sc[...] += jnp.sum(o, axis=0, keepdims=True)
            s_sc[...] += jnp.sum(o * o, axis=0, keepdims=True)

        @pl.when(p == 1)
        def _():
            mean = m_sc[...] * inv_n
            var = s_sc[...] * inv_n - mean * mean
            scale = g_ref[...] / jnp.sqrt(var + _BN_EPS)
            shift = b_ref[...] - mean * scale
            y = jax.nn.relu(o * scale + shift)
            o_ref[0] = jnp.transpose(y, (1, 0)).reshape(_CH, _SROWS, _BEV_W)

    nrow = _NCELL // _BEV_W
    return pl.pallas_call(
        body,
        grid=(2, B, _NSTRIP),
        in_specs=[
            pl.BlockSpec((1, _STRIP, _CH), lambda p, b, i: (b, i, 0)),
            pl.BlockSpec((1, _BEV_W, _CH),
                         lambda p, b, i: (b, jnp.maximum(i * _SROWS - 1, 0), 0)),
            pl.BlockSpec((1, _BEV_W, _CH),
                         lambda p, b, i: (b, jnp.minimum((i + 1) * _SROWS, nrow - 1), 0)),
            pl.BlockSpec((3, 3, _CH, _CH), lambda p, b, i: (0, 0, 0, 0)),
            pl.BlockSpec((1, _CH), lambda p, b, i: (0, 0)),
            pl.BlockSpec((1, _CH), lambda p, b, i: (0, 0)),
        ],
        out_specs=pl.BlockSpec((1, _CH, _SROWS, _BEV_W), lambda p, b, i: (b, 0, i, 0)),
        out_shape=jax.ShapeDtypeStruct((B, _CH, _BEV_H, _BEV_W), jnp.float32),
        scratch_shapes=[
            pltpu.VMEM((1, _CH), jnp.float32),
            pltpu.VMEM((1, _CH), jnp.float32),
        ],
    )


def _conv2d(x, w, bias=None, padding=0, groups=1):
    out = lax.conv_general_dilated(
        x, w, window_strides=(1, 1),
        padding=[(padding, padding), (padding, padding)],
        feature_group_count=groups,
        dimension_numbers=("NCHW", "OIHW", "NCHW"))
    if bias is not None:
        out = out + bias[None, :, None, None]
    return out


def _bnorm(x, g, b, eps=1e-5):
    m = jnp.mean(x, axis=(0, 2, 3), keepdims=True)
    v = jnp.var(x, axis=(0, 2, 3), keepdims=True)
    return (x - m) / jnp.sqrt(v + eps) * g[None, :, None, None] + b[None, :, None, None]


def _pixel_grid(H, W):
    x = jnp.linspace(0.0, W - 1.0, W)
    y = jnp.linspace(0.0, H - 1.0, H)
    yy, xx = jnp.meshgrid(y, x, indexing="ij")
    return jnp.stack([xx, yy, jnp.ones_like(xx)], axis=-1).reshape(-1, 3).T


def kernel(feat_stage1, feat_stage2, feat_stage3, feat_stage5, intrinsics, extrinsics, params):
    feats = {"stage1": feat_stage1, "stage2": feat_stage2, "stage3": feat_stage3, "stage5": feat_stage5}
    Bsz = feat_stage1.shape[0]
    depth_bins = jnp.exp(jnp.linspace(jnp.log(1.0), jnp.log(60.0), _DEPTH_CH))
    K_inv = jnp.linalg.inv(intrinsics)

    accs = {}
    for s in _SCALES:
        p = params[s]
        f = feats[s]
        H, W = f.shape[2], f.shape[3]
        N = H * W

        h = jax.nn.relu(_bnorm(_conv2d(f, p["dn_w1"]), p["dn_g1"], p["dn_b1"]))
        depth_logits = _conv2d(h, p["dn_w2"], bias=p["dn_bias2"])
        dl_pm = jnp.transpose(depth_logits.reshape(Bsz, 64, H * W), (0, 2, 1))
        depth_probs_pm = jax.nn.softmax(dl_pm * 10.0, axis=2)
        depth_map = jnp.sum(depth_probs_pm * depth_bins[None, None, :], axis=2)
        pixels = jnp.broadcast_to(_pixel_grid(H, W)[None], (Bsz, 3, H * W))
        dm = depth_map.reshape(Bsz, 1, -1)
        cam_pts = dm * jnp.einsum("bij,bjn->bin", K_inv, pixels)
        cam_pts_h = jnp.concatenate([cam_pts, jnp.ones_like(cam_pts[:, :1])], axis=1)
        ego = jnp.einsum("bij,bjn->bin", extrinsics, cam_pts_h)[:, :3]
        bev_x = (ego[:, 0] / _VOX[0] + _BEV_W // 2).astype(jnp.int32)
        bev_y = (ego[:, 1] / _VOX[1] + _BEV_H // 2).astype(jnp.int32)
        valid = (bev_x >= 0) & (bev_x < _BEV_W) & (bev_y >= 0) & (bev_y < _BEV_H)
        idx = jnp.where(valid, bev_y * _BEV_W + bev_x, 0)

        C = f.shape[1]
        u1, st_u = _make_fr1(C, H, W, Bsz)(
            f.reshape(Bsz, C, N), p["fr_w1"].reshape(64, C))
        w2d = jnp.zeros((_CH, 64, 3, 3), jnp.float32)
        for g in range(8):
            w2d = w2d.at[g * 16:(g + 1) * 16, g * 8:(g + 1) * 8].set(
                p["fr_w2"][g * 16:(g + 1) * 16])
        w2d = jnp.transpose(w2d, (2, 3, 1, 0))
        v, st_v = _make_fr2(H, W, Bsz)(
            u1, u1, u1, st_u, p["fr_g1"].reshape(1, 64), p["fr_b1"].reshape(1, 64), w2d)

        dlp = jnp.transpose(depth_logits.reshape(Bsz, 64, N), (0, 2, 1))
        wdl = jnp.transpose(p["cn_w1"][:, :64], (2, 3, 1, 0))
        wrd = jnp.transpose(p["cn_w1"][:, 64:], (2, 3, 1, 0))
        g2 = p["fr_g2"].reshape(1, _CH)
        b2 = p["fr_b2"].reshape(1, _CH)
        wp, st_w = _make_cn1(H, W, Bsz)(
            dlp, dlp, dlp, v, v, v, st_v, g2, b2, wdl, wrd)

        rows = _make_rows(H, W, Bsz)(
            wp, st_w, p["cn_g1"].reshape(1, 64), p["cn_b1"].reshape(1, 64),
            p["cn_w2"].reshape(1, 64), p["cn_bias2"].reshape(1, 1),
            v, st_v, g2, b2,
            valid.astype(jnp.float32).reshape(Bsz, N, 1))
        accs[s] = _make_sc_scatter(H * W, Bsz)(idx.astype(jnp.int32), rows)

    outs = []
    for s in _SKIP:
        sp = params["sp_" + s]
        co = sp["w"].shape[0]
        outs.append(_make_skip_head(co, Bsz)(
            accs[s], sp["w"].reshape(co, _CH),
            sp["g"].reshape(1, co), sp["b"].reshape(1, co)))
    mp = params["mp"]
    main = _make_main_head(Bsz)(
        accs["stage5"], accs["stage5"], accs["stage5"],
        jnp.transpose(mp["w"], (2, 3, 1, 0)),
        mp["g"].reshape(1, _CH), mp["b"].reshape(1, _CH))
    return (main, outs[0], outs[1], outs[2])


# speedup vs baseline: 16.6072x; 1.0872x over previous
import functools

import jax
import jax.numpy as jnp
from jax import lax
from jax.experimental import pallas as pl
from jax.experimental.pallas import tpu as pltpu
from jax.experimental.pallas import tpu_sc as plsc

_SCALES = ["stage1", "stage2", "stage3", "stage5"]
_HWS = {"stage1": (128, 128), "stage2": (64, 64), "stage3": (32, 32), "stage5": (16, 16)}
_SKIP = ["stage1", "stage2", "stage3"]
_DEPTH_CH = 64
_BEV_H = 128
_BEV_W = 128
_NCELL = _BEV_H * _BEV_W
_VOX = (0.4, 0.4)

_CH = 128
_HALF = _NCELL // 2
_DUMP = _HALF
_ACC_ROWS = _HALF + 128
_ZROWS = 130


@functools.cache
def _make_sc_scatter(N, B):
    mesh = plsc.VectorSubcoreMesh(core_axis_name="c", subcore_axis_name="s")
    chunk = N // 16
    K = min(128, chunk)
    nk = chunk // K

    @functools.partial(
        pl.kernel,
        mesh=mesh,
        out_type=jax.ShapeDtypeStruct((B, _NCELL, _CH), jnp.float32),
        scratch_types=[
            pltpu.VMEM_SHARED((_ACC_ROWS, _CH), jnp.float32),
            pltpu.VMEM((K,), jnp.int32),
            pltpu.VMEM((K,), jnp.int32),
            pltpu.VMEM((K, _CH), jnp.float32),
            pltpu.VMEM((K, _CH), jnp.float32),
            pltpu.VMEM((_ZROWS, _CH), jnp.float32),
            pltpu.SemaphoreType.DMA,
            pltpu.SemaphoreType.DMA,
            pltpu.SemaphoreType.DMA,
            pltpu.SemaphoreType.DMA,
            pltpu.SemaphoreType.DMA,
        ],
    )
    def kfn(idx_hbm, rows_hbm, acc_hbm, acc_sh, idx0, idx1, rows0, rows1,
            zbuf, sl0, sl1, sc0, sc1, sw):
        c = lax.axis_index("c")
        s = lax.axis_index("s")
        half_base = c * _HALF
        idx_v = [idx0, idx1]
        rows_v = [rows0, rows1]
        sem_ld = [sl0, sl1]
        sem_sc = [sc0, sc1]

        def zrow(i, _):
            def zcol(j, __):
                zbuf[i, pl.ds(j * 16, 16)] = jnp.zeros((16,), jnp.float32)
                return 0
            return lax.fori_loop(0, _CH // 16, zcol, 0)
        lax.fori_loop(0, _ZROWS, zrow, 0)

        def load(b, t, p):
            off = s * chunk + t * K
            ha = pltpu.async_copy(idx_hbm.at[b, pl.ds(off, K)], idx_v[p], sem_ld[p])
            hb = pltpu.async_copy(rows_hbm.at[b, pl.ds(off, K)], rows_v[p], sem_ld[p])
            return ha, hb

        for b in range(B):
            hz = [pltpu.async_copy(
                      zbuf, acc_sh.at[pl.ds(s * 4 * _ZROWS + j * _ZROWS, _ZROWS)], sw)
                  for j in range(4)]
            for h in hz:
                h.wait()
            plsc.subcore_barrier()

            pend = load(b, 0, 0)
            hsc = [None, None]
            for t in range(nk):
                p = t % 2
                for h in pend:
                    h.wait()
                for v in range(K // 16):
                    t16 = idx_v[p][pl.ds(v * 16, 16)]
                    loc = t16 - half_base
                    ok = (loc >= 0) & (loc < _HALF)
                    idx_v[p][pl.ds(v * 16, 16)] = jnp.where(ok, loc, _DUMP)
                hsc[p] = pltpu.async_copy(rows_v[p], acc_sh.at[idx_v[p]],
                                          sem_sc[p], add=True)
                if t + 1 < nk:
                    if hsc[1 - p] is not None:
                        hsc[1 - p].wait()
                        hsc[1 - p] = None
                    pend = load(b, t + 1, 1 - p)
            for p in range(2):
                if hsc[p] is not None:
                    hsc[p].wait()
            plsc.subcore_barrier()

            hw = [pltpu.async_copy(
                      acc_sh.at[pl.ds(s * 512 + j * 128, 128)],
                      acc_hbm.at[b, pl.ds(half_base + s * 512 + j * 128, 128)], sw)
                  for j in range(4)]
            for h in hw:
                h.wait()
            plsc.subcore_barrier()

    return kfn


def _strip_cfg(H, W):
    srows = min(16, H)
    return srows, srows * W, H // srows


def _conv3x3_taps(xbs, wv, strip, W, CO):
    wq = jax.lax.broadcasted_iota(jnp.int32, (strip, 1), 0) % W
    o = jnp.zeros((strip, CO), jnp.float32)
    for kx in range(3):
        a = jnp.zeros((strip, CO), jnp.float32)
        for ky in range(3):
            for xb, w in zip(xbs, wv):
                src = jax.lax.slice(xb, (ky * W, 0), (ky * W + strip, xb.shape[1]))
                a = a + jax.lax.dot_general(src, w[ky, kx], (((1,), (0,)), ((), ())),
                                            preferred_element_type=jnp.float32)
        if kx == 0:
            sh = jnp.concatenate([jnp.zeros((1, CO), jnp.float32), a[:-1]], axis=0)
            o = o + sh * (wq >= 1).astype(jnp.float32)
        elif kx == 1:
            o = o + a
        else:
            sh = jnp.concatenate([a[1:], jnp.zeros((1, CO), jnp.float32)], axis=0)
            o = o + sh * (wq <= W - 2).astype(jnp.float32)
    return o


def _halo_specs(H, W, CI, srows, strip):
    return [
        pl.BlockSpec((1, strip, CI), lambda b, i: (b, i, 0)),
        pl.BlockSpec((1, W, CI), lambda b, i: (b, jnp.maximum(i * srows - 1, 0), 0)),
        pl.BlockSpec((1, W, CI), lambda b, i: (b, jnp.minimum((i + 1) * srows, H - 1), 0)),
    ]


def _edge_halo(up, dn, i, ns):
    above = up * (i > 0).astype(up.dtype)
    below = dn * (i < ns - 1).astype(dn.dtype)
    return above, below


@functools.cache
def _make_fr1(C, H, W, B):
    srows, strip, ns = _strip_cfg(H, W)

    def body(f_ref, w_ref, u_ref, st_ref):
        b = pl.program_id(0)
        i = pl.program_id(1)

        @pl.when(jnp.logical_and(b == 0, i == 0))
        def _():
            st_ref[...] = jnp.zeros_like(st_ref)

        fb = f_ref[0].astype(jnp.bfloat16)
        wb = w_ref[...].astype(jnp.bfloat16)
        o = jax.lax.dot_general(wb, fb, (((1,), (0,)), ((), ())),
                                preferred_element_type=jnp.float32)
        st_ref[0, :] += jnp.sum(o, axis=1)
        st_ref[1, :] += jnp.sum(o * o, axis=1)
        u_ref[0] = jnp.transpose(o, (1, 0))

    return pl.pallas_call(
        body,
        grid=(B, ns),
        in_specs=[
            pl.BlockSpec((1, C, strip), lambda b, i: (b, 0, i)),
            pl.BlockSpec((64, C), lambda b, i: (0, 0)),
        ],
        out_specs=[
            pl.BlockSpec((1, strip, 64), lambda b, i: (b, i, 0)),
            pl.BlockSpec((2, 64), lambda b, i: (0, 0)),
        ],
        out_shape=[
            jax.ShapeDtypeStruct((B, H * W, 64), jnp.float32),
            jax.ShapeDtypeStruct((2, 64), jnp.float32),
        ],
    )


@functools.cache
def _make_fr2(H, W, B):
    srows, strip, ns = _strip_cfg(H, W)
    inv_n = 1.0 / (B * H * W)

    def body(x_ref, up_ref, dn_ref, st_u_ref, g_ref, bb_ref, w_ref, v_ref, st_ref):
        b = pl.program_id(0)
        i = pl.program_id(1)

        @pl.when(jnp.logical_and(b == 0, i == 0))
        def _():
            st_ref[...] = jnp.zeros_like(st_ref)

        mean = st_u_ref[0:1, :] * inv_n
        var = st_u_ref[1:2, :] * inv_n - mean * mean
        scale = g_ref[...] / jnp.sqrt(var + _BN_EPS)
        shift = bb_ref[...] - mean * scale

        def r1(x):
            return jax.nn.relu(x * scale + shift).astype(jnp.bfloat16)

        above, below = _edge_halo(r1(up_ref[0]), r1(dn_ref[0]), i, ns)
        xb = jnp.concatenate([above, r1(x_ref[0]), below], axis=0)
        o = _conv3x3_taps([xb], [w_ref[...].astype(jnp.bfloat16)], strip, W, _CH)
        v_ref[0] = o
        st_ref[0, :] += jnp.sum(o, axis=0)
        st_ref[1, :] += jnp.sum(o * o, axis=0)

    return pl.pallas_call(
        body,
        grid=(B, ns),
        in_specs=_halo_specs(H, W, 64, srows, strip) + [
            pl.BlockSpec((2, 64), lambda b, i: (0, 0)),
            pl.BlockSpec((1, 64), lambda b, i: (0, 0)),
            pl.BlockSpec((1, 64), lambda b, i: (0, 0)),
            pl.BlockSpec((3, 3, 64, _CH), lambda b, i: (0, 0, 0, 0)),
        ],
        out_specs=[
            pl.BlockSpec((1, strip, _CH), lambda b, i: (b, i, 0)),
            pl.BlockSpec((2, _CH), lambda b, i: (0, 0)),
        ],
        out_shape=[
            jax.ShapeDtypeStruct((B, H * W, _CH), jnp.float32),
            jax.ShapeDtypeStruct((2, _CH), jnp.float32),
        ],
    )


@functools.cache
def _make_cn1(H, W, B):
    srows, strip, ns = _strip_cfg(H, W)
    inv_n = 1.0 / (B * H * W)

    def body(dl_ref, dlu_ref, dld_ref, v_ref, vu_ref, vd_ref,
             st_v_ref, g_ref, bb_ref, wdl_ref, wrd_ref, o_ref, st_ref):
        b = pl.program_id(0)
        i = pl.program_id(1)

        @pl.when(jnp.logical_and(b == 0, i == 0))
        def _():
            st_ref[...] = jnp.zeros_like(st_ref)

        mean = st_v_ref[0:1, :] * inv_n
        var = st_v_ref[1:2, :] * inv_n - mean * mean
        scale = g_ref[...] / jnp.sqrt(var + _BN_EPS)
        shift = bb_ref[...] - mean * scale

        def red(x):
            return jax.nn.relu(x * scale + shift).astype(jnp.bfloat16)

        vau, vbd = _edge_halo(red(vu_ref[0]), red(vd_ref[0]), i, ns)
        vb = jnp.concatenate([vau, red(v_ref[0]), vbd], axis=0)
        dau, dbd = _edge_halo(dlu_ref[0].astype(jnp.bfloat16),
                              dld_ref[0].astype(jnp.bfloat16), i, ns)
        db = jnp.concatenate([dau, dl_ref[0].astype(jnp.bfloat16), dbd], axis=0)

        o = _conv3x3_taps([db, vb],
                          [wdl_ref[...].astype(jnp.bfloat16),
                           wrd_ref[...].astype(jnp.bfloat16)], strip, W, 64)
        o_ref[0] = o
        st_ref[0, :] += jnp.sum(o, axis=0)
        st_ref[1, :] += jnp.sum(o * o, axis=0)

    return pl.pallas_call(
        body,
        grid=(B, ns),
        in_specs=(_halo_specs(H, W, 64, srows, strip)
                  + _halo_specs(H, W, _CH, srows, strip)
                  + [
                      pl.BlockSpec((2, _CH), lambda b, i: (0, 0)),
                      pl.BlockSpec((1, _CH), lambda b, i: (0, 0)),
                      pl.BlockSpec((1, _CH), lambda b, i: (0, 0)),
                      pl.BlockSpec((3, 3, 64, 64), lambda b, i: (0, 0, 0, 0)),
                      pl.BlockSpec((3, 3, _CH, 64), lambda b, i: (0, 0, 0, 0)),
                  ]),
        out_specs=[
            pl.BlockSpec((1, strip, 64), lambda b, i: (b, i, 0)),
            pl.BlockSpec((2, 64), lambda b, i: (0, 0)),
        ],
        out_shape=[
            jax.ShapeDtypeStruct((B, H * W, 64), jnp.float32),
            jax.ShapeDtypeStruct((2, 64), jnp.float32),
        ],
    )


@functools.cache
def _make_rows(H, W, B):
    srows, strip, ns = _strip_cfg(H, W)
    inv_n = 1.0 / (B * H * W)

    def body(wp_ref, st_w_ref, cg_ref, cb_ref, w2_ref, b2_ref,
             v_ref, st_v_ref, g_ref, bb_ref, valid_ref, o_ref):
        meanw = st_w_ref[0:1, :] * inv_n
        varw = st_w_ref[1:2, :] * inv_n - meanw * meanw
        scw = cg_ref[...] / jnp.sqrt(varw + _BN_EPS)
        shw = cb_ref[...] - meanw * scw
        c = jax.nn.relu(wp_ref[0] * scw + shw).astype(jnp.bfloat16)
        w2 = w2_ref[...].astype(jnp.bfloat16).astype(jnp.float32)
        logit = jnp.sum(c.astype(jnp.float32) * w2, axis=1, keepdims=True)
        conf = jax.nn.sigmoid(logit + b2_ref[...])

        meanv = st_v_ref[0:1, :] * inv_n
        varv = st_v_ref[1:2, :] * inv_n - meanv * meanv
        scv = g_ref[...] / jnp.sqrt(varv + _BN_EPS)
        shv = bb_ref[...] - meanv * scv
        reduced = jax.nn.relu(v_ref[0] * scv + shv)
        o_ref[0] = reduced * conf * valid_ref[0, :, 0:1]

    return pl.pallas_call(
        body,
        grid=(B, ns),
        in_specs=[
            pl.BlockSpec((1, strip, 64), lambda b, i: (b, i, 0)),
            pl.BlockSpec((2, 64), lambda b, i: (0, 0)),
            pl.BlockSpec((1, 64), lambda b, i: (0, 0)),
            pl.BlockSpec((1, 64), lambda b, i: (0, 0)),
            pl.BlockSpec((1, 64), lambda b, i: (0, 0)),
            pl.BlockSpec((1, 1), lambda b, i: (0, 0)),
            pl.BlockSpec((1, strip, _CH), lambda b, i: (b, i, 0)),
            pl.BlockSpec((2, _CH), lambda b, i: (0, 0)),
            pl.BlockSpec((1, _CH), lambda b, i: (0, 0)),
            pl.BlockSpec((1, _CH), lambda b, i: (0, 0)),
            pl.BlockSpec((1, strip, 1), lambda b, i: (b, i, 0)),
        ],
        out_specs=pl.BlockSpec((1, strip, _CH), lambda b, i: (b, i, 0)),
        out_shape=jax.ShapeDtypeStruct((B, H * W, _CH), jnp.float32),
    )


_BN_EPS = 1e-5


_NSTRIP = 8
_STRIP = _NCELL // _NSTRIP
_SROWS = _STRIP // _BEV_W


@functools.cache
def _make_skip_head(co, B):
    inv_n = 1.0 / (B * _NCELL)

    def body(x_ref, w_ref, g_ref, b_ref, o_ref, m_sc, s_sc, ost):
        p = pl.program_id(0)
        b = pl.program_id(1)
        i = pl.program_id(2)

        @pl.when(jnp.logical_and(p == 0, jnp.logical_and(b == 0, i == 0)))
        def _():
            m_sc[...] = jnp.zeros_like(m_sc)
            s_sc[...] = jnp.zeros_like(s_sc)

        @pl.when(p == 0)
        def _():
            xb = x_ref[0].astype(jnp.bfloat16)
            wb = w_ref[...].astype(jnp.bfloat16)
            o = jax.lax.dot_general(xb, wb, (((1,), (1,)), ((), ())),
                                    preferred_element_type=jnp.float32)
            m_sc[...] += jnp.sum(o, axis=0, keepdims=True)
            s_sc[...] += jnp.sum(o * o, axis=0, keepdims=True)
            ost[b, i] = o

        @pl.when(p == 1)
        def _():
            o = ost[b, i]
            mean = m_sc[...] * inv_n
            var = s_sc[...] * inv_n - mean * mean
            scale = g_ref[...] / jnp.sqrt(var + _BN_EPS)
            shift = b_ref[...] - mean * scale
            y = jax.nn.relu(o * scale + shift)
            o_ref[0] = jnp.transpose(y, (1, 0)).reshape(co, _SROWS, _BEV_W)

    return pl.pallas_call(
        body,
        grid=(2, B, _NSTRIP),
        in_specs=[
            pl.BlockSpec((1, _STRIP, _CH),
                         lambda p, b, i: (jnp.where(p == 0, b, 0),
                                          jnp.where(p == 0, i, 0), 0)),
            pl.BlockSpec((co, _CH), lambda p, b, i: (0, 0)),
            pl.BlockSpec((1, co), lambda p, b, i: (0, 0)),
            pl.BlockSpec((1, co), lambda p, b, i: (0, 0)),
        ],
        out_specs=pl.BlockSpec((1, co, _SROWS, _BEV_W), lambda p, b, i: (b, 0, i, 0)),
        out_shape=jax.ShapeDtypeStruct((B, co, _BEV_H, _BEV_W), jnp.float32),
        scratch_shapes=[
            pltpu.VMEM((1, co), jnp.float32),
            pltpu.VMEM((1, co), jnp.float32),
            pltpu.VMEM((B, _NSTRIP, _STRIP, co), jnp.float32),
        ],
    )


@functools.cache
def _make_main_head(B):
    inv_n = 1.0 / (B * _NCELL)

    def body(x_ref, up_ref, dn_ref, w_ref, g_ref, b_ref, o_ref, m_sc, s_sc, ost):
        p = pl.program_id(0)
        b = pl.program_id(1)
        i = pl.program_id(2)

        @pl.when(jnp.logical_and(p == 0, jnp.logical_and(b == 0, i == 0)))
        def _():
            m_sc[...] = jnp.zeros_like(m_sc)
            s_sc[...] = jnp.zeros_like(s_sc)

        @pl.when(p == 0)
        def _():
            above = up_ref[0].astype(jnp.bfloat16) * (i > 0).astype(jnp.bfloat16)
            below = dn_ref[0].astype(jnp.bfloat16) * (i < _NSTRIP - 1).astype(jnp.bfloat16)
            xb = jnp.concatenate([above, x_ref[0].astype(jnp.bfloat16), below], axis=0)

            wv = w_ref[...].astype(jnp.bfloat16)
            wq = jax.lax.broadcasted_iota(jnp.int32, (_STRIP, 1), 0) % _BEV_W
            o = jnp.zeros((_STRIP, _CH), jnp.float32)
            for kx in range(3):
                a = jnp.zeros((_STRIP, _CH), jnp.float32)
                for ky in range(3):
                    src = jax.lax.slice(xb, (ky * _BEV_W, 0), (ky * _BEV_W + _STRIP, _CH))
                    a = a + jax.lax.dot_general(src, wv[ky, kx], (((1,), (0,)), ((), ())),
                                                preferred_element_type=jnp.float32)
                if kx == 0:
                    sh = jnp.concatenate([jnp.zeros((1, _CH), jnp.float32), a[:-1]], axis=0)
                    o = o + sh * (wq >= 1).astype(jnp.float32)
                elif kx == 1:
                    o = o + a
                else:
                    sh = jnp.concatenate([a[1:], jnp.zeros((1, _CH), jnp.float32)], axis=0)
                    o = o + sh * (wq <= _BEV_W - 2).astype(jnp.float32)
            m_sc[...] += jnp.sum(o, axis=0, keepdims=True)
            s_sc[...] += jnp.sum(o * o, axis=0, keepdims=True)
            ost[b, i] = o

        @pl.when(p == 1)
        def _():
            o = ost[b, i]
            mean = m_sc[...] * inv_n
            var = s_sc[...] * inv_n - mean * mean
            scale = g_ref[...] / jnp.sqrt(var + _BN_EPS)
            shift = b_ref[...] - mean * scale
            y = jax.nn.relu(o * scale + shift)
            o_ref[0] = jnp.transpose(y, (1, 0)).reshape(_CH, _SROWS, _BEV_W)

    nrow = _NCELL // _BEV_W
    return pl.pallas_call(
        body,
        grid=(2, B, _NSTRIP),
        in_specs=[
            pl.BlockSpec((1, _STRIP, _CH),
                         lambda p, b, i: (jnp.where(p == 0, b, 0),
                                          jnp.where(p == 0, i, 0), 0)),
            pl.BlockSpec((1, _BEV_W, _CH),
                         lambda p, b, i: (jnp.where(p == 0, b, 0),
                                          jnp.maximum(i * _SROWS - 1, 0) * (p == 0), 0)),
            pl.BlockSpec((1, _BEV_W, _CH),
                         lambda p, b, i: (jnp.where(p == 0, b, 0),
                                          jnp.minimum((i + 1) * _SROWS, nrow - 1) * (p == 0), 0)),
            pl.BlockSpec((3, 3, _CH, _CH), lambda p, b, i: (0, 0, 0, 0)),
            pl.BlockSpec((1, _CH), lambda p, b, i: (0, 0)),
            pl.BlockSpec((1, _CH), lambda p, b, i: (0, 0)),
        ],
        out_specs=pl.BlockSpec((1, _CH, _SROWS, _BEV_W), lambda p, b, i: (b, 0, i, 0)),
        out_shape=jax.ShapeDtypeStruct((B, _CH, _BEV_H, _BEV_W), jnp.float32),
        scratch_shapes=[
            pltpu.VMEM((1, _CH), jnp.float32),
            pltpu.VMEM((1, _CH), jnp.float32),
            pltpu.VMEM((B, _NSTRIP, _STRIP, _CH), jnp.float32),
        ],
    )


def _conv2d(x, w, bias=None, padding=0, groups=1):
    out = lax.conv_general_dilated(
        x, w, window_strides=(1, 1),
        padding=[(padding, padding), (padding, padding)],
        feature_group_count=groups,
        dimension_numbers=("NCHW", "OIHW", "NCHW"))
    if bias is not None:
        out = out + bias[None, :, None, None]
    return out


def _bnorm(x, g, b, eps=1e-5):
    m = jnp.mean(x, axis=(0, 2, 3), keepdims=True)
    v = jnp.var(x, axis=(0, 2, 3), keepdims=True)
    return (x - m) / jnp.sqrt(v + eps) * g[None, :, None, None] + b[None, :, None, None]


def _pixel_grid(H, W):
    x = jnp.linspace(0.0, W - 1.0, W)
    y = jnp.linspace(0.0, H - 1.0, H)
    yy, xx = jnp.meshgrid(y, x, indexing="ij")
    return jnp.stack([xx, yy, jnp.ones_like(xx)], axis=-1).reshape(-1, 3).T


def kernel(feat_stage1, feat_stage2, feat_stage3, feat_stage5, intrinsics, extrinsics, params):
    feats = {"stage1": feat_stage1, "stage2": feat_stage2, "stage3": feat_stage3, "stage5": feat_stage5}
    Bsz = feat_stage1.shape[0]
    depth_bins = jnp.exp(jnp.linspace(jnp.log(1.0), jnp.log(60.0), _DEPTH_CH))
    K_inv = jnp.linalg.inv(intrinsics)

    accs = {}
    for s in _SCALES:
        p = params[s]
        f = feats[s]
        H, W = f.shape[2], f.shape[3]
        N = H * W

        h = jax.nn.relu(_bnorm(_conv2d(f, p["dn_w1"]), p["dn_g1"], p["dn_b1"]))
        depth_logits = _conv2d(h, p["dn_w2"], bias=p["dn_bias2"])
        dl_pm = jnp.transpose(depth_logits.reshape(Bsz, 64, H * W), (0, 2, 1))
        depth_probs_pm = jax.nn.softmax(dl_pm * 10.0, axis=2)
        depth_map = jnp.sum(depth_probs_pm * depth_bins[None, None, :], axis=2)
        pixels = jnp.broadcast_to(_pixel_grid(H, W)[None], (Bsz, 3, H * W))
        dm = depth_map.reshape(Bsz, 1, -1)
        cam_pts = dm * jnp.einsum("bij,bjn->bin", K_inv, pixels)
        cam_pts_h = jnp.concatenate([cam_pts, jnp.ones_like(cam_pts[:, :1])], axis=1)
        ego = jnp.einsum("bij,bjn->bin", extrinsics, cam_pts_h)[:, :3]
        bev_x = (ego[:, 0] / _VOX[0] + _BEV_W // 2).astype(jnp.int32)
        bev_y = (ego[:, 1] / _VOX[1] + _BEV_H // 2).astype(jnp.int32)
        valid = (bev_x >= 0) & (bev_x < _BEV_W) & (bev_y >= 0) & (bev_y < _BEV_H)
        idx = jnp.where(valid, bev_y * _BEV_W + bev_x, 0)

        C = f.shape[1]
        u1, st_u = _make_fr1(C, H, W, Bsz)(
            f.reshape(Bsz, C, N), p["fr_w1"].reshape(64, C))
        w2d = jnp.zeros((_CH, 64, 3, 3), jnp.float32)
        for g in range(8):
            w2d = w2d.at[g * 16:(g + 1) * 16, g * 8:(g + 1) * 8].set(
                p["fr_w2"][g * 16:(g + 1) * 16])
        w2d = jnp.transpose(w2d, (2, 3, 1, 0))
        v, st_v = _make_fr2(H, W, Bsz)(
            u1, u1, u1, st_u, p["fr_g1"].reshape(1, 64), p["fr_b1"].reshape(1, 64), w2d)

        dlp = jnp.transpose(depth_logits.reshape(Bsz, 64, N), (0, 2, 1))
        wdl = jnp.transpose(p["cn_w1"][:, :64], (2, 3, 1, 0))
        wrd = jnp.transpose(p["cn_w1"][:, 64:], (2, 3, 1, 0))
        g2 = p["fr_g2"].reshape(1, _CH)
        b2 = p["fr_b2"].reshape(1, _CH)
        wp, st_w = _make_cn1(H, W, Bsz)(
            dlp, dlp, dlp, v, v, v, st_v, g2, b2, wdl, wrd)

        rows = _make_rows(H, W, Bsz)(
            wp, st_w, p["cn_g1"].reshape(1, 64), p["cn_b1"].reshape(1, 64),
            p["cn_w2"].reshape(1, 64), p["cn_bias2"].reshape(1, 1),
            v, st_v, g2, b2,
            valid.astype(jnp.float32).reshape(Bsz, N, 1))
        accs[s] = _make_sc_scatter(H * W, Bsz)(idx.astype(jnp.int32), rows)

    outs = []
    for s in _SKIP:
        sp = params["sp_" + s]
        co = sp["w"].shape[0]
        outs.append(_make_skip_head(co, Bsz)(
            accs[s], sp["w"].reshape(co, _CH),
            sp["g"].reshape(1, co), sp["b"].reshape(1, co)))
    mp = params["mp"]
    main = _make_main_head(Bsz)(
        accs["stage5"], accs["stage5"], accs["stage5"],
        jnp.transpose(mp["w"], (2, 3, 1, 0)),
        mp["g"].reshape(1, _CH), mp["b"].reshape(1, _CH))
    return (main, outs[0], outs[1], outs[2])
